# Initial kernel scaffold; baseline (speedup 1.0000x reference)
import jax
import jax.numpy as jnp
from jax.experimental import pallas as pl


def kernel(params, x_main, ei_main, ea_main, fcw_main, batch_main, x1, ei1, ea1, fcw1, batch1, x2, ei2, ea2, fcw2, batch2):
    raise NotImplementedError("write your pallas kernel here")


# speedup vs baseline: 1.5737x; 1.5737x over previous
import functools

import jax
import jax.numpy as jnp
from jax import lax
from jax.experimental import pallas as pl
from jax.experimental.pallas import tpu as pltpu
from jax.experimental.pallas import tpu_sc as plsc

D = 128
N_MAIN = 10000
NM_P = 10112
NB = 8192
ROW_B1 = NM_P
ROW_B2 = NM_P + NB
N_H = NM_P + 2 * NB + 128
E_MAIN = 160000
E_MAIN_P = 163840
EB = 65536
E_ALL = E_MAIN_P + 2 * EB
BLOCK = 512
K_SUB = 16

_SC_NC = 2
_SC_NS = 16

_PHASES = (
    (0, E_MAIN_P // 128 // 32, NM_P, 0, True),
    (E_MAIN_P, EB // 128 // 32, NB, ROW_B1, False),
    (E_MAIN_P + EB, EB // 128 // 32, NB + 128, ROW_B2, False),
)
_NSP = NM_P


def _edge_sc(p_mat, q_mat, src_g, dst_l):
    mesh = plsc.VectorSubcoreMesh(
        core_axis_name="c", subcore_axis_name="s",
        num_cores=_SC_NC, num_subcores=_SC_NS)

    @functools.partial(
        pl.kernel,
        out_type=jax.ShapeDtypeStruct((_SC_NC, N_H, D), jnp.float32),
        mesh=mesh,
        scratch_types=[
            pltpu.VMEM((128,), jnp.int32),
            pltpu.VMEM((128,), jnp.int32),
            pltpu.VMEM((128, D), jnp.float32),
            pltpu.VMEM((128, D), jnp.float32),
            pltpu.VMEM((128, D), jnp.float32),
            pltpu.VMEM_SHARED((_NSP, D), jnp.float32),
            pltpu.SemaphoreType.DMA,
        ],
    )
    def k(p_hbm, q_hbm, src_hbm, dst_hbm, out_hbm,
          srcv, dstv, prow, qrow, zbuf, agg, sem):
        c = lax.axis_index("c")
        s = lax.axis_index("s")
        w = s * _SC_NC + c

        def zb(r, carry):
            for j in range(D // 16):
                zbuf[r, pl.ds(16 * j, 16)] = jnp.zeros((16,), jnp.float32)
            return carry
        lax.fori_loop(0, 128, zb, 0)

        for (e0, cpw, n_loc, row_base, has_q) in _PHASES:
            nch = n_loc // 128

            def zc(i, carry):
                cid = s + i * _SC_NS
                pltpu.sync_copy(zbuf, agg.at[pl.ds(cid * 128, 128)])
                return carry
            lax.fori_loop(0, (nch - s + _SC_NS - 1) // _SC_NS, zc, 0)
            plsc.subcore_barrier()

            def ec(kk, carry):
                base = e0 + (w * cpw + kk) * 128
                pltpu.sync_copy(src_hbm.at[pl.ds(base, 128)], srcv)
                pltpu.sync_copy(dst_hbm.at[pl.ds(base, 128)], dstv)
                cp = pltpu.async_copy(p_hbm.at[srcv], prow, sem)
                if has_q:
                    pltpu.sync_copy(q_hbm.at[pl.ds(base, 128)], qrow)
                cp.wait()

                def cb(r, inner):
                    for j in range(D // 16):
                        sl = pl.ds(16 * j, 16)
                        v = prow[r, sl]
                        if has_q:
                            v = v + qrow[r, sl]
                        prow[r, sl] = jnp.maximum(v, 0.0)
                    return inner
                lax.fori_loop(0, 128, cb, 0)

                pltpu.sync_copy(prow, agg.at[dstv], add=True)
                return carry
            lax.fori_loop(0, cpw, ec, 0)
            plsc.subcore_barrier()

            def co(i, carry):
                cid = s + i * _SC_NS
                pltpu.sync_copy(
                    agg.at[pl.ds(cid * 128, 128)],
                    out_hbm.at[c, pl.ds(row_base + cid * 128, 128)])
                return carry
            lax.fori_loop(0, (nch - s + _SC_NS - 1) // _SC_NS, co, 0)
            plsc.subcore_barrier()

    return k(p_mat, q_mat, src_g, dst_l)


def _mm_relu_body(x_ref, w_ref, b_ref, o_ref):
    o_ref[:] = jnp.maximum(
        jnp.dot(x_ref[:], w_ref[:], preferred_element_type=jnp.float32)
        + b_ref[:], 0.0)


def _mm_relu(x, w, b):
    m = x.shape[0]
    return pl.pallas_call(
        _mm_relu_body,
        out_shape=jax.ShapeDtypeStruct((m, D), jnp.float32),
        grid=(m // 128,),
        in_specs=[
            pl.BlockSpec((128, x.shape[1]), lambda i: (i, 0)),
            pl.BlockSpec((x.shape[1], D), lambda i: (0, 0)),
            pl.BlockSpec((1, D), lambda i: (0, 0)),
        ],
        out_specs=pl.BlockSpec((128, D), lambda i: (i, 0)),
    )(x, w, b)


def _mm_bias_body(x_ref, w_ref, b_ref, o_ref):
    o_ref[:] = (jnp.dot(x_ref[:], w_ref[:], preferred_element_type=jnp.float32)
                + b_ref[:])


def _q_proj(ea, w, b):
    m = ea.shape[0]
    return pl.pallas_call(
        _mm_bias_body,
        out_shape=jax.ShapeDtypeStruct((m, D), jnp.float32),
        grid=(m // 256,),
        in_specs=[
            pl.BlockSpec((256, ea.shape[1]), lambda i: (i, 0)),
            pl.BlockSpec((ea.shape[1], D), lambda i: (0, 0)),
            pl.BlockSpec((1, D), lambda i: (0, 0)),
        ],
        out_specs=pl.BlockSpec((256, D), lambda i: (i, 0)),
    )(ea, w, b)


def _p_body(h_ref, w_ref, qt_ref, o_ref):
    o_ref[:] = (jnp.dot(h_ref[:], w_ref[:], preferred_element_type=jnp.float32)
                + qt_ref[0])


def _qtab_sel(i):
    b1 = ROW_B1 // 128
    b2 = ROW_B2 // 128
    end = (ROW_B2 + NB) // 128
    return jnp.where(
        i < b1, 0,
        jnp.where(i < b2, 1 + (i - b1) % 4,
                  jnp.where(i < end, 5 + (i - b2) % 4, 0)))


def _p_proj(h, w, qtab):
    return pl.pallas_call(
        _p_body,
        out_shape=jax.ShapeDtypeStruct((N_H, D), jnp.float32),
        grid=(N_H // 128,),
        in_specs=[
            pl.BlockSpec((128, D), lambda i: (i, 0)),
            pl.BlockSpec((D, D), lambda i: (0, 0)),
            pl.BlockSpec((1, 128, D), lambda i: (_qtab_sel(i), 0, 0)),
        ],
        out_specs=pl.BlockSpec((128, D), lambda i: (i, 0)),
    )(h, w, qtab)


def _upd_body(h_ref, ap_ref, wh_ref, wa_ref, b_ref, o_ref):
    a = ap_ref[0] + ap_ref[1]
    o_ref[:] = jnp.maximum(
        jnp.dot(h_ref[:], wh_ref[:], preferred_element_type=jnp.float32)
        + jnp.dot(a, wa_ref[:], preferred_element_type=jnp.float32)
        + b_ref[:], 0.0)


def _update(h, aggp, wh, wa, b):
    return pl.pallas_call(
        _upd_body,
        out_shape=jax.ShapeDtypeStruct((N_H, D), jnp.float32),
        grid=(N_H // 128,),
        in_specs=[
            pl.BlockSpec((128, D), lambda i: (i, 0)),
            pl.BlockSpec((2, 128, D), lambda i: (0, i, 0)),
            pl.BlockSpec((D, D), lambda i: (0, 0)),
            pl.BlockSpec((D, D), lambda i: (0, 0)),
            pl.BlockSpec((1, D), lambda i: (0, 0)),
        ],
        out_specs=pl.BlockSpec((128, D), lambda i: (i, 0)),
    )(h, aggp, wh, wa, b)


def _ln(x, s, b):
    m = jnp.mean(x, axis=-1, keepdims=True)
    v = jnp.mean((x - m) ** 2, axis=-1, keepdims=True)
    return (x - m) * lax.rsqrt(v + 1e-5) * s + b


def _dot(a, b):
    return jnp.dot(a, b, preferred_element_type=jnp.float32)


def _tail_body(h_ref, fcw_ref,
               wq_ref, wk_ref, wv_ref, wo_ref,
               bq_ref, bk_ref, bv_ref, bo_ref,
               l1s_ref, l1b_ref, l2s_ref, l2b_ref,
               w1_ref, b1_ref, w2_ref, b2_ref,
               fuw_ref, fub_ref, fua_ref,
               cw1_ref, cb1_ref, ca_ref, cw2_ref, cb2_ref,
               o_ref):
    wgt = h_ref[:] * fcw_ref[:]
    me = jnp.sum(wgt[0:N_MAIN], axis=0, keepdims=True)
    ap1 = jnp.sum(wgt[ROW_B1:ROW_B1 + NB].reshape(K_SUB, BLOCK, D), axis=1)
    ap2 = jnp.sum(wgt[ROW_B2:ROW_B2 + NB].reshape(K_SUB, BLOCK, D), axis=1)

    def prelu(x, a):
        return jnp.where(x >= 0, x, a * x)

    def mha(q, kv, l):
        lq = q.shape[0]
        qp = _dot(q, wq_ref[l]) + bq_ref[l]
        kp = _dot(kv, wk_ref[l]) + bk_ref[l]
        vp = _dot(kv, wv_ref[l]) + bv_ref[l]
        outs = []
        for hh in range(8):
            sl = slice(16 * hh, 16 * hh + 16)
            att = lax.dot_general(qp[:, sl], kp[:, sl],
                                  (((1,), (1,)), ((), ())),
                                  preferred_element_type=jnp.float32) * 0.25
            att = att - jnp.max(att, axis=-1, keepdims=True)
            att = jnp.exp(att)
            att = att / jnp.sum(att, axis=-1, keepdims=True)
            outs.append(_dot(att, vp[:, sl]))
        o = jnp.concatenate(outs, axis=1)
        return _dot(o, wo_ref[l]) + bo_ref[l]

    def enc(x, kv, l):
        x = _ln(x + mha(x, kv, l), l1s_ref[l], l1b_ref[l])
        f = _dot(jnp.maximum(_dot(x, w1_ref[l]) + b1_ref[l], 0.0), w2_ref[l])
        return _ln(x + f + b2_ref[l], l2s_ref[l], l2b_ref[l])

    def branch(ap, g, lbase):
        fw = fuw_ref[g]
        ap = prelu(_dot(ap, fw[:D]) + _dot(me, fw[D:]) + fub_ref[g],
                   fua_ref[g])
        ap = enc(ap, ap, lbase)
        ap = enc(ap, ap, lbase + 1)
        q = enc(me, ap, lbase + 2)
        q = enc(q, ap, lbase + 3)
        return q

    c1 = branch(ap1, 0, 0)
    c2 = branch(ap2, 1, 4)
    ci = jnp.concatenate([me, c1, c2], axis=1)
    z = prelu(_dot(ci, cw1_ref[:]) + cb1_ref[:], ca_ref[0])
    z = _dot(z, cw2_ref[:]) + cb2_ref[:]
    o_ref[:] = 1.0 / (1.0 + jnp.exp(-z))


def _tail(h, fcw, aw, fu, clf):
    ins = [h, fcw] + aw + fu + clf
    return pl.pallas_call(
        _tail_body,
        out_shape=jax.ShapeDtypeStruct((1, 256), jnp.float32),
    )(*ins)


def kernel(params, x_main, ei_main, ea_main, fcw_main, batch_main,
           x1, ei1, ea1, fcw1, batch1, x2, ei2, ea2, fcw2, batch2):
    gnn = params['gnn']
    f32 = jnp.float32

    zpad_m = jnp.zeros((NM_P - N_MAIN, D), f32)
    zpad_t = jnp.zeros((128, D), f32)
    x_all = jnp.concatenate([x_main, zpad_m, x1, x2, zpad_t], axis=0)
    fcw_all = jnp.concatenate([
        fcw_main, jnp.zeros((NM_P - N_MAIN,), f32),
        fcw1, fcw2, jnp.zeros((128,), f32)])[:, None]

    ea_all = jnp.concatenate([
        ea_main, ea1[:BLOCK], ea2[:BLOCK],
        jnp.zeros((E_MAIN_P - E_MAIN - 2 * BLOCK, ea_main.shape[1]), f32)],
        axis=0)

    pad_e = E_MAIN_P - E_MAIN
    src_g = jnp.concatenate([
        ei_main[0], jnp.zeros((pad_e,), jnp.int32),
        ei1[0] + ROW_B1, ei2[0] + ROW_B2])
    dst_l = jnp.concatenate([
        ei_main[1], jnp.full((pad_e,), N_MAIN, jnp.int32),
        ei1[1], ei2[1]])

    h = _mm_relu(x_all, gnn['W0'], gnn['b0'][None])

    for lp in gnn['layers']:
        wm, bm = lp['Wm'], lp['bm']
        q_mat = _q_proj(ea_all, wm[D:], bm[None])
        qtab = jnp.concatenate([
            jnp.zeros((1, 128, D), f32),
            q_mat[E_MAIN:E_MAIN + BLOCK].reshape(4, 128, D),
            q_mat[E_MAIN + BLOCK:E_MAIN + 2 * BLOCK].reshape(4, 128, D)],
            axis=0)
        p_mat = _p_proj(h, wm[:D], qtab)
        aggp = _edge_sc(p_mat, q_mat, src_g, dst_l)
        h = _update(h, aggp, lp['Wu'][:D], lp['Wu'][D:], lp['bu'][None])

    layers = (params['sa1'] + params['cr1'] + params['sa2'] + params['cr2'])

    def stack(name):
        return jnp.stack([l[name] for l in layers])

    aw = [stack('Wq'), stack('Wk'), stack('Wv'), stack('Wo'),
          stack('bq'), stack('bk'), stack('bv'), stack('bo'),
          stack('ln1_s'), stack('ln1_b'), stack('ln2_s'), stack('ln2_b'),
          stack('W1'), stack('b1'), stack('W2'), stack('b2')]
    fu = [jnp.stack([params['fusion']['W'], params['fusion2']['W']]),
          jnp.stack([params['fusion']['b'], params['fusion2']['b']]),
          jnp.stack([params['fusion']['a'], params['fusion2']['a']])[:, None]]
    cp = params['clf']
    clf = [cp['W1'], cp['b1'][None], jnp.reshape(cp['a'], (1, 1)),
           cp['W2'], cp['b2'][None]]

    return _tail(h, fcw_all, aw, fu, clf)


# speedup vs baseline: 1.8149x; 1.1533x over previous
import functools

import jax
import jax.numpy as jnp
from jax import lax
from jax.experimental import pallas as pl
from jax.experimental.pallas import tpu as pltpu
from jax.experimental.pallas import tpu_sc as plsc

D = 128
N_MAIN = 10000
NM_P = 10112
NB = 8192
ROW_B1 = NM_P
ROW_B2 = NM_P + NB
N_H = NM_P + 2 * NB + 128
E_MAIN = 160000
E_MAIN_P = 163840
EB = 65536
E_ALL = E_MAIN_P + 2 * EB
BLOCK = 512
K_SUB = 16

_SC_NC = 2
_SC_NS = 16

_PHASES = (
    (0, E_MAIN_P, NM_P, 0, True),
    (E_MAIN_P, EB, NB, ROW_B1, False),
    (E_MAIN_P + EB, EB, NB + 128, ROW_B2, False),
)
_NSP = NM_P


_CH = 64


def _edge_sc(p_mat, q_mat, sd_pack):
    mesh = plsc.VectorSubcoreMesh(
        core_axis_name="c", subcore_axis_name="s",
        num_cores=_SC_NC, num_subcores=_SC_NS)

    @functools.partial(
        pl.kernel,
        out_type=jax.ShapeDtypeStruct((_SC_NC, N_H, D), jnp.float32),
        mesh=mesh,
        scratch_types=[
            pltpu.VMEM((2, _CH), jnp.int32),
            pltpu.VMEM((2, _CH), jnp.int32),
            pltpu.VMEM((_CH,), jnp.int32),
            pltpu.VMEM((_CH,), jnp.int32),
            pltpu.VMEM((_CH, D), jnp.float32),
            pltpu.VMEM((_CH, D), jnp.float32),
            pltpu.VMEM((_CH, D), jnp.float32),
            pltpu.VMEM((_CH, D), jnp.float32),
            pltpu.VMEM((_CH, D), jnp.float32),
            pltpu.VMEM((_CH, D), jnp.float32),
            pltpu.VMEM_SHARED((_NSP, D), jnp.float32),
            pltpu.SemaphoreType.DMA,
            pltpu.SemaphoreType.DMA,
            pltpu.SemaphoreType.DMA,
            pltpu.SemaphoreType.DMA,
            pltpu.SemaphoreType.DMA,
            pltpu.SemaphoreType.DMA,
            pltpu.SemaphoreType.DMA,
            pltpu.SemaphoreType.DMA,
        ],
    )
    def k(p_hbm, q_hbm, sd_hbm, out_hbm,
          sd0, sd1, dc0, dc1, mr0, mr1, qr0, qr1, sb0, sb1, agg,
          is0, is1, qs0, qs1, gs0, gs1, ss0, ss1):
        c = lax.axis_index("c")
        s = lax.axis_index("s")
        w = s * _SC_NC + c
        sd = (sd0, sd1)
        dc = (dc0, dc1)
        mr = (mr0, mr1)
        qr = (qr0, qr1)
        sb = (sb0, sb1)
        isem = (is0, is1)
        qsem = (qs0, qs1)
        gsem = (gs0, gs1)
        ssem = (ss0, ss1)

        def drain(sem, dst_ref, dummy_src):
            pltpu.make_async_copy(dummy_src, dst_ref, sem).wait()

        def zero_qr0():
            def zb(r, carry):
                for j in range(D // 16):
                    qr0[r, pl.ds(16 * j, 16)] = jnp.zeros((16,), jnp.float32)
                return carry
            lax.fori_loop(0, _CH, zb, 0)

        first = True
        for (e0, cpe, n_loc, row_base, has_q) in _PHASES:
            cpw = cpe // _CH // 32
            nch = n_loc // _CH
            ch0 = e0 // _CH

            if first or not has_q:
                zero_qr0()
            first = False

            def zc(i, carry):
                cid = s + i * _SC_NS
                pltpu.sync_copy(qr0, agg.at[pl.ds(cid * _CH, _CH)])
                return carry
            lax.fori_loop(0, (nch - s + _SC_NS - 1) // _SC_NS, zc, 0)
            plsc.subcore_barrier()

            def fire_idx(kk, b):
                cid = ch0 + w * cpw + kk
                pltpu.async_copy(sd_hbm.at[cid], sd[b], isem[b])

            def fire_q(kk, b):
                cid = ch0 + w * cpw + kk
                pltpu.async_copy(q_hbm.at[pl.ds(cid * _CH, _CH)],
                                 qr[b], qsem[b])

            def fire_gather(b):
                drain(isem[b], sd[b], sd_hbm.at[0])
                pltpu.async_copy(p_hbm.at[sd[b].at[0]], mr[b], gsem[b])

            def step(kk, b, drain_ssem, prefetch):
                drain(gsem[b], mr[b], p_hbm.at[pl.ds(0, _CH)])
                if drain_ssem:
                    drain(ssem[b], sb[b], p_hbm.at[pl.ds(0, _CH)])
                for j in range(_CH // 16):
                    dc[b][pl.ds(16 * j, 16)] = sd[b][1, pl.ds(16 * j, 16)]
                if prefetch:
                    fire_idx(kk + 2, b)
                if has_q:
                    drain(qsem[b], qr[b], p_hbm.at[pl.ds(0, _CH)])

                def cb(r, carry):
                    for u in range(2):
                        for j in range(D // 16):
                            sl = pl.ds(16 * j, 16)
                            v = mr[b][2 * r + u, sl]
                            if has_q:
                                v = v + qr[b][2 * r + u, sl]
                            sb[b][2 * r + u, sl] = jnp.maximum(v, 0.0)
                    return carry
                lax.fori_loop(0, _CH // 2, cb, 0)

                pltpu.async_copy(sb[b], agg.at[dc[b]], ssem[b], add=True)
                if prefetch:
                    if has_q:
                        fire_q(kk + 2, b)
                    fire_gather(b)

            fire_idx(0, 0)
            fire_idx(1, 1)
            if has_q:
                fire_q(0, 0)
                fire_q(1, 1)
            fire_gather(0)
            fire_gather(1)
            step(0, 0, False, True)
            step(1, 1, False, True)

            def ms(i, carry):
                step(2 + 2 * i, 0, True, True)
                step(3 + 2 * i, 1, True, True)
                return carry
            lax.fori_loop(0, (cpw - 4) // 2, ms, 0)

            step(cpw - 2, 0, True, False)
            step(cpw - 1, 1, True, False)
            drain(ssem[0], sb[0], p_hbm.at[pl.ds(0, _CH)])
            drain(ssem[1], sb[1], p_hbm.at[pl.ds(0, _CH)])
            plsc.subcore_barrier()

            def co(i, carry):
                cid = s + i * _SC_NS
                pltpu.sync_copy(
                    agg.at[pl.ds(cid * _CH, _CH)],
                    out_hbm.at[c, pl.ds(row_base + cid * _CH, _CH)])
                return carry
            lax.fori_loop(0, (nch - s + _SC_NS - 1) // _SC_NS, co, 0)
            plsc.subcore_barrier()

    return k(p_mat, q_mat, sd_pack)


def _mm_relu_body(x_ref, w_ref, b_ref, o_ref):
    o_ref[:] = jnp.maximum(
        jnp.dot(x_ref[:], w_ref[:], preferred_element_type=jnp.float32)
        + b_ref[:], 0.0)


def _mm_relu(x, w, b):
    m = x.shape[0]
    return pl.pallas_call(
        _mm_relu_body,
        out_shape=jax.ShapeDtypeStruct((m, D), jnp.float32),
        grid=(m // 128,),
        in_specs=[
            pl.BlockSpec((128, x.shape[1]), lambda i: (i, 0)),
            pl.BlockSpec((x.shape[1], D), lambda i: (0, 0)),
            pl.BlockSpec((1, D), lambda i: (0, 0)),
        ],
        out_specs=pl.BlockSpec((128, D), lambda i: (i, 0)),
    )(x, w, b)


def _mm_bias_body(x_ref, w_ref, b_ref, o_ref):
    o_ref[:] = (jnp.dot(x_ref[:], w_ref[:], preferred_element_type=jnp.float32)
                + b_ref[:])


def _q_proj(ea, w, b):
    m = ea.shape[0]
    return pl.pallas_call(
        _mm_bias_body,
        out_shape=jax.ShapeDtypeStruct((m, D), jnp.float32),
        grid=(m // 256,),
        in_specs=[
            pl.BlockSpec((256, ea.shape[1]), lambda i: (i, 0)),
            pl.BlockSpec((ea.shape[1], D), lambda i: (0, 0)),
            pl.BlockSpec((1, D), lambda i: (0, 0)),
        ],
        out_specs=pl.BlockSpec((256, D), lambda i: (i, 0)),
    )(ea, w, b)


def _p_body(h_ref, w_ref, qt_ref, o_ref):
    o_ref[:] = (jnp.dot(h_ref[:], w_ref[:], preferred_element_type=jnp.float32)
                + qt_ref[0])


def _qtab_sel(i):
    b1 = ROW_B1 // 128
    b2 = ROW_B2 // 128
    end = (ROW_B2 + NB) // 128
    return jnp.where(
        i < b1, 0,
        jnp.where(i < b2, 1 + (i - b1) % 4,
                  jnp.where(i < end, 5 + (i - b2) % 4, 0)))


def _p_proj(h, w, qtab):
    return pl.pallas_call(
        _p_body,
        out_shape=jax.ShapeDtypeStruct((N_H, D), jnp.float32),
        grid=(N_H // 128,),
        in_specs=[
            pl.BlockSpec((128, D), lambda i: (i, 0)),
            pl.BlockSpec((D, D), lambda i: (0, 0)),
            pl.BlockSpec((1, 128, D), lambda i: (_qtab_sel(i), 0, 0)),
        ],
        out_specs=pl.BlockSpec((128, D), lambda i: (i, 0)),
    )(h, w, qtab)


def _upd_body(h_ref, ap_ref, wh_ref, wa_ref, b_ref, o_ref):
    a = ap_ref[0] + ap_ref[1]
    o_ref[:] = jnp.maximum(
        jnp.dot(h_ref[:], wh_ref[:], preferred_element_type=jnp.float32)
        + jnp.dot(a, wa_ref[:], preferred_element_type=jnp.float32)
        + b_ref[:], 0.0)


def _update(h, aggp, wh, wa, b):
    return pl.pallas_call(
        _upd_body,
        out_shape=jax.ShapeDtypeStruct((N_H, D), jnp.float32),
        grid=(N_H // 128,),
        in_specs=[
            pl.BlockSpec((128, D), lambda i: (i, 0)),
            pl.BlockSpec((2, 128, D), lambda i: (0, i, 0)),
            pl.BlockSpec((D, D), lambda i: (0, 0)),
            pl.BlockSpec((D, D), lambda i: (0, 0)),
            pl.BlockSpec((1, D), lambda i: (0, 0)),
        ],
        out_specs=pl.BlockSpec((128, D), lambda i: (i, 0)),
    )(h, aggp, wh, wa, b)


def _ln(x, s, b):
    m = jnp.mean(x, axis=-1, keepdims=True)
    v = jnp.mean((x - m) ** 2, axis=-1, keepdims=True)
    return (x - m) * lax.rsqrt(v + 1e-5) * s + b


def _dot(a, b):
    return jnp.dot(a, b, preferred_element_type=jnp.float32)


def _tail_body(h_ref, fcw_ref,
               wq_ref, wk_ref, wv_ref, wo_ref,
               bq_ref, bk_ref, bv_ref, bo_ref,
               l1s_ref, l1b_ref, l2s_ref, l2b_ref,
               w1_ref, b1_ref, w2_ref, b2_ref,
               fuw_ref, fub_ref, fua_ref,
               cw1_ref, cb1_ref, ca_ref, cw2_ref, cb2_ref,
               o_ref):
    wgt = h_ref[:] * fcw_ref[:]
    me = jnp.sum(wgt[0:N_MAIN], axis=0, keepdims=True)
    ap1 = jnp.sum(wgt[ROW_B1:ROW_B1 + NB].reshape(K_SUB, BLOCK, D), axis=1)
    ap2 = jnp.sum(wgt[ROW_B2:ROW_B2 + NB].reshape(K_SUB, BLOCK, D), axis=1)

    def prelu(x, a):
        return jnp.where(x >= 0, x, a * x)

    def mha(q, kv, l):
        lq = q.shape[0]
        qp = _dot(q, wq_ref[l]) + bq_ref[l]
        kp = _dot(kv, wk_ref[l]) + bk_ref[l]
        vp = _dot(kv, wv_ref[l]) + bv_ref[l]
        outs = []
        for hh in range(8):
            sl = slice(16 * hh, 16 * hh + 16)
            att = lax.dot_general(qp[:, sl], kp[:, sl],
                                  (((1,), (1,)), ((), ())),
                                  preferred_element_type=jnp.float32) * 0.25
            att = att - jnp.max(att, axis=-1, keepdims=True)
            att = jnp.exp(att)
            att = att / jnp.sum(att, axis=-1, keepdims=True)
            outs.append(_dot(att, vp[:, sl]))
        o = jnp.concatenate(outs, axis=1)
        return _dot(o, wo_ref[l]) + bo_ref[l]

    def enc(x, kv, l):
        x = _ln(x + mha(x, kv, l), l1s_ref[l], l1b_ref[l])
        f = _dot(jnp.maximum(_dot(x, w1_ref[l]) + b1_ref[l], 0.0), w2_ref[l])
        return _ln(x + f + b2_ref[l], l2s_ref[l], l2b_ref[l])

    def branch(ap, g, lbase):
        fw = fuw_ref[g]
        ap = prelu(_dot(ap, fw[:D]) + _dot(me, fw[D:]) + fub_ref[g],
                   fua_ref[g])
        ap = enc(ap, ap, lbase)
        ap = enc(ap, ap, lbase + 1)
        q = enc(me, ap, lbase + 2)
        q = enc(q, ap, lbase + 3)
        return q

    c1 = branch(ap1, 0, 0)
    c2 = branch(ap2, 1, 4)
    ci = jnp.concatenate([me, c1, c2], axis=1)
    z = prelu(_dot(ci, cw1_ref[:]) + cb1_ref[:], ca_ref[0])
    z = _dot(z, cw2_ref[:]) + cb2_ref[:]
    o_ref[:] = 1.0 / (1.0 + jnp.exp(-z))


def _tail(h, fcw, aw, fu, clf):
    ins = [h, fcw] + aw + fu + clf
    return pl.pallas_call(
        _tail_body,
        out_shape=jax.ShapeDtypeStruct((1, 256), jnp.float32),
    )(*ins)


def kernel(params, x_main, ei_main, ea_main, fcw_main, batch_main,
           x1, ei1, ea1, fcw1, batch1, x2, ei2, ea2, fcw2, batch2):
    gnn = params['gnn']
    f32 = jnp.float32

    zpad_m = jnp.zeros((NM_P - N_MAIN, D), f32)
    zpad_t = jnp.zeros((128, D), f32)
    x_all = jnp.concatenate([x_main, zpad_m, x1, x2, zpad_t], axis=0)
    fcw_all = jnp.concatenate([
        fcw_main, jnp.zeros((NM_P - N_MAIN,), f32),
        fcw1, fcw2, jnp.zeros((128,), f32)])[:, None]

    ea_all = jnp.concatenate([
        ea_main, ea1[:BLOCK], ea2[:BLOCK],
        jnp.zeros((E_MAIN_P - E_MAIN - 2 * BLOCK, ea_main.shape[1]), f32)],
        axis=0)

    pad_e = E_MAIN_P - E_MAIN
    src_g = jnp.concatenate([
        ei_main[0], jnp.zeros((pad_e,), jnp.int32),
        ei1[0] + ROW_B1, ei2[0] + ROW_B2])
    dst_l = jnp.concatenate([
        ei_main[1], jnp.full((pad_e,), N_MAIN, jnp.int32),
        ei1[1], ei2[1]])
    sd_pack = jnp.stack(
        [src_g.reshape(-1, _CH), dst_l.reshape(-1, _CH)], axis=1)

    h = _mm_relu(x_all, gnn['W0'], gnn['b0'][None])

    for lp in gnn['layers']:
        wm, bm = lp['Wm'], lp['bm']
        q_mat = _q_proj(ea_all, wm[D:], bm[None])
        qtab = jnp.concatenate([
            jnp.zeros((1, 128, D), f32),
            q_mat[E_MAIN:E_MAIN + BLOCK].reshape(4, 128, D),
            q_mat[E_MAIN + BLOCK:E_MAIN + 2 * BLOCK].reshape(4, 128, D)],
            axis=0)
        p_mat = _p_proj(h, wm[:D], qtab)
        aggp = _edge_sc(p_mat, q_mat, sd_pack)
        h = _update(h, aggp, lp['Wu'][:D], lp['Wu'][D:], lp['bu'][None])

    layers = (params['sa1'] + params['cr1'] + params['sa2'] + params['cr2'])

    def stack(name):
        return jnp.stack([l[name] for l in layers])

    aw = [stack('Wq'), stack('Wk'), stack('Wv'), stack('Wo'),
          stack('bq'), stack('bk'), stack('bv'), stack('bo'),
          stack('ln1_s'), stack('ln1_b'), stack('ln2_s'), stack('ln2_b'),
          stack('W1'), stack('b1'), stack('W2'), stack('b2')]
    fu = [jnp.stack([params['fusion']['W'], params['fusion2']['W']]),
          jnp.stack([params['fusion']['b'], params['fusion2']['b']]),
          jnp.stack([params['fusion']['a'], params['fusion2']['a']])[:, None]]
    cp = params['clf']
    clf = [cp['W1'], cp['b1'][None], jnp.reshape(cp['a'], (1, 1)),
           cp['W2'], cp['b2'][None]]

    return _tail(h, fcw_all, aw, fu, clf)


# speedup vs baseline: 3.2573x; 1.7947x over previous
import functools

import jax
import jax.numpy as jnp
from jax import lax
from jax.experimental import pallas as pl
from jax.experimental.pallas import tpu as pltpu
from jax.experimental.pallas import tpu_sc as plsc

D = 128
N_MAIN = 10000
NM_P = 10240
NB = 8192
ROW_B1 = NM_P
ROW_B2 = NM_P + NB
N_H = NM_P + 2 * NB
E_MAIN = 160000
E_MAIN_P = 163840
EB = 65536
E_ALL = E_MAIN_P + 2 * EB
BLOCK = 512
K_SUB = 16

_SC_NC = 2
_SC_NS = 16

_NAGG = 10048

_PHASES = (
    (0, E_MAIN_P, _NAGG, 0, True),
    (E_MAIN_P, EB, NB, ROW_B1, False),
    (E_MAIN_P + EB, EB, NB, ROW_B2, False),
)
_NSP = _NAGG


_CH = 64


def _edge_sc(p_mat, q_mat, sd_pack):
    mesh = plsc.VectorSubcoreMesh(
        core_axis_name="c", subcore_axis_name="s",
        num_cores=_SC_NC, num_subcores=_SC_NS)

    @functools.partial(
        pl.kernel,
        out_type=jax.ShapeDtypeStruct((_SC_NC, N_H, D), jnp.float32),
        mesh=mesh,
        scratch_types=[
            pltpu.VMEM((2, _CH), jnp.int32),
            pltpu.VMEM((2, _CH), jnp.int32),
            pltpu.VMEM((_CH,), jnp.int32),
            pltpu.VMEM((_CH,), jnp.int32),
            pltpu.VMEM((_CH, D), jnp.float32),
            pltpu.VMEM((_CH, D), jnp.float32),
            pltpu.VMEM((_CH, D), jnp.float32),
            pltpu.VMEM((_CH, D), jnp.float32),
            pltpu.VMEM((_CH, D), jnp.float32),
            pltpu.VMEM((_CH, D), jnp.float32),
            pltpu.VMEM_SHARED((_NSP, D), jnp.float32),
            pltpu.SemaphoreType.DMA,
            pltpu.SemaphoreType.DMA,
            pltpu.SemaphoreType.DMA,
            pltpu.SemaphoreType.DMA,
            pltpu.SemaphoreType.DMA,
            pltpu.SemaphoreType.DMA,
            pltpu.SemaphoreType.DMA,
            pltpu.SemaphoreType.DMA,
        ],
    )
    def k(p_hbm, q_hbm, sd_hbm, out_hbm,
          sd0, sd1, dc0, dc1, mr0, mr1, qr0, qr1, sb0, sb1, agg,
          is0, is1, qs0, qs1, gs0, gs1, ss0, ss1):
        c = lax.axis_index("c")
        s = lax.axis_index("s")
        w = s * _SC_NC + c
        sd = (sd0, sd1)
        dc = (dc0, dc1)
        mr = (mr0, mr1)
        qr = (qr0, qr1)
        sb = (sb0, sb1)
        isem = (is0, is1)
        qsem = (qs0, qs1)
        gsem = (gs0, gs1)
        ssem = (ss0, ss1)

        def drain(sem, dst_ref, dummy_src):
            pltpu.make_async_copy(dummy_src, dst_ref, sem).wait()

        def zero_qr0():
            def zb(r, carry):
                for j in range(D // 16):
                    qr0[r, pl.ds(16 * j, 16)] = jnp.zeros((16,), jnp.float32)
                return carry
            lax.fori_loop(0, _CH, zb, 0)

        first = True
        for (e0, cpe, n_loc, row_base, has_q) in _PHASES:
            cpw = cpe // _CH // 32
            nch = n_loc // _CH
            ch0 = e0 // _CH

            if first or not has_q:
                zero_qr0()
            first = False

            def zc(i, carry):
                cid = s + i * _SC_NS
                pltpu.sync_copy(qr0, agg.at[pl.ds(cid * _CH, _CH)])
                return carry
            lax.fori_loop(0, (nch - s + _SC_NS - 1) // _SC_NS, zc, 0)
            plsc.subcore_barrier()

            def fire_idx(kk, b):
                cid = ch0 + w * cpw + kk
                pltpu.async_copy(sd_hbm.at[cid], sd[b], isem[b])

            def fire_q(kk, b):
                cid = ch0 + w * cpw + kk
                pltpu.async_copy(q_hbm.at[pl.ds(cid * _CH, _CH)],
                                 qr[b], qsem[b])

            def fire_gather(b):
                drain(isem[b], sd[b], sd_hbm.at[0])
                pltpu.async_copy(p_hbm.at[sd[b].at[0]], mr[b], gsem[b])

            def step(kk, b, drain_ssem, prefetch):
                drain(gsem[b], mr[b], p_hbm.at[pl.ds(0, _CH)])
                if drain_ssem:
                    drain(ssem[b], sb[b], p_hbm.at[pl.ds(0, _CH)])
                for j in range(_CH // 16):
                    dc[b][pl.ds(16 * j, 16)] = sd[b][1, pl.ds(16 * j, 16)]
                if prefetch:
                    fire_idx(kk + 2, b)
                if has_q:
                    drain(qsem[b], qr[b], p_hbm.at[pl.ds(0, _CH)])

                def cb(r, carry):
                    for u in range(4):
                        for j in range(D // 16):
                            sl = pl.ds(16 * j, 16)
                            v = mr[b][4 * r + u, sl]
                            if has_q:
                                v = v + qr[b][4 * r + u, sl]
                            sb[b][4 * r + u, sl] = jnp.maximum(v, 0.0)
                    return carry
                lax.fori_loop(0, _CH // 4, cb, 0)

                pltpu.async_copy(sb[b], agg.at[dc[b]], ssem[b], add=True)
                if prefetch:
                    if has_q:
                        fire_q(kk + 2, b)
                    fire_gather(b)

            fire_idx(0, 0)
            fire_idx(1, 1)
            if has_q:
                fire_q(0, 0)
                fire_q(1, 1)
            fire_gather(0)
            fire_gather(1)
            step(0, 0, False, True)
            step(1, 1, False, True)

            def ms(i, carry):
                step(2 + 2 * i, 0, True, True)
                step(3 + 2 * i, 1, True, True)
                return carry
            lax.fori_loop(0, (cpw - 4) // 2, ms, 0)

            step(cpw - 2, 0, True, False)
            step(cpw - 1, 1, True, False)
            drain(ssem[0], sb[0], p_hbm.at[pl.ds(0, _CH)])
            drain(ssem[1], sb[1], p_hbm.at[pl.ds(0, _CH)])
            plsc.subcore_barrier()

            def co(i, carry):
                cid = s + i * _SC_NS
                pltpu.sync_copy(
                    agg.at[pl.ds(cid * _CH, _CH)],
                    out_hbm.at[c, pl.ds(row_base + cid * _CH, _CH)])
                return carry
            lax.fori_loop(0, (nch - s + _SC_NS - 1) // _SC_NS, co, 0)
            plsc.subcore_barrier()

    return k(p_mat, q_mat, sd_pack)


def _mm_relu_body(x_ref, w_ref, b_ref, o_ref):
    o_ref[:] = jnp.maximum(
        jnp.dot(x_ref[:], w_ref[:], preferred_element_type=jnp.float32)
        + b_ref[:], 0.0)


def _mm_relu(x, w, b):
    m = x.shape[0]
    return pl.pallas_call(
        _mm_relu_body,
        out_shape=jax.ShapeDtypeStruct((m, D), jnp.float32),
        grid=(m // 512,),
        in_specs=[
            pl.BlockSpec((512, x.shape[1]), lambda i: (i, 0)),
            pl.BlockSpec((x.shape[1], D), lambda i: (0, 0)),
            pl.BlockSpec((1, D), lambda i: (0, 0)),
        ],
        out_specs=pl.BlockSpec((512, D), lambda i: (i, 0)),
    )(x, w, b)


def _mm_bias_body(x_ref, w_ref, b_ref, o_ref):
    o_ref[:] = (jnp.dot(x_ref[:], w_ref[:], preferred_element_type=jnp.float32)
                + b_ref[:])


def _q_proj(ea, w, b):
    m = ea.shape[0]
    return pl.pallas_call(
        _mm_bias_body,
        out_shape=jax.ShapeDtypeStruct((m, D), jnp.float32),
        grid=(m // 2048,),
        in_specs=[
            pl.BlockSpec((2048, ea.shape[1]), lambda i: (i, 0)),
            pl.BlockSpec((ea.shape[1], D), lambda i: (0, 0)),
            pl.BlockSpec((1, D), lambda i: (0, 0)),
        ],
        out_specs=pl.BlockSpec((2048, D), lambda i: (i, 0)),
    )(ea, w, b)


def _p_body(h_ref, w_ref, qt_ref, o_ref):
    o_ref[:] = (jnp.dot(h_ref[:], w_ref[:], preferred_element_type=jnp.float32)
                + qt_ref[0])


def _qtab_sel(i):
    b1 = ROW_B1 // 512
    b2 = ROW_B2 // 512
    return jnp.where(i < b1, 0, jnp.where(i < b2, 1, 2))


def _p_proj(h, w, qtab):
    return pl.pallas_call(
        _p_body,
        out_shape=jax.ShapeDtypeStruct((N_H, D), jnp.float32),
        grid=(N_H // 512,),
        in_specs=[
            pl.BlockSpec((512, D), lambda i: (i, 0)),
            pl.BlockSpec((D, D), lambda i: (0, 0)),
            pl.BlockSpec((1, 512, D), lambda i: (_qtab_sel(i), 0, 0)),
        ],
        out_specs=pl.BlockSpec((512, D), lambda i: (i, 0)),
    )(h, w, qtab)


def _upd_body(h_ref, ap_ref, wh_ref, wa_ref, b_ref, o_ref):
    a = ap_ref[0] + ap_ref[1]
    o_ref[:] = jnp.maximum(
        jnp.dot(h_ref[:], wh_ref[:], preferred_element_type=jnp.float32)
        + jnp.dot(a, wa_ref[:], preferred_element_type=jnp.float32)
        + b_ref[:], 0.0)


def _update(h, aggp, wh, wa, b):
    return pl.pallas_call(
        _upd_body,
        out_shape=jax.ShapeDtypeStruct((N_H, D), jnp.float32),
        grid=(N_H // 512,),
        in_specs=[
            pl.BlockSpec((512, D), lambda i: (i, 0)),
            pl.BlockSpec((2, 512, D), lambda i: (0, i, 0)),
            pl.BlockSpec((D, D), lambda i: (0, 0)),
            pl.BlockSpec((D, D), lambda i: (0, 0)),
            pl.BlockSpec((1, D), lambda i: (0, 0)),
        ],
        out_specs=pl.BlockSpec((512, D), lambda i: (i, 0)),
    )(h, aggp, wh, wa, b)


def _ln(x, s, b):
    m = jnp.mean(x, axis=-1, keepdims=True)
    v = jnp.mean((x - m) ** 2, axis=-1, keepdims=True)
    return (x - m) * lax.rsqrt(v + 1e-5) * s + b


def _dot(a, b):
    return jnp.dot(a, b, preferred_element_type=jnp.float32)


def _tail_body(h_ref, fcw_ref,
               wq_ref, wk_ref, wv_ref, wo_ref,
               bq_ref, bk_ref, bv_ref, bo_ref,
               l1s_ref, l1b_ref, l2s_ref, l2b_ref,
               w1_ref, b1_ref, w2_ref, b2_ref,
               fuw_ref, fub_ref, fua_ref,
               cw1_ref, cb1_ref, ca_ref, cw2_ref, cb2_ref,
               o_ref):
    wgt = h_ref[:] * fcw_ref[:]
    me = jnp.sum(wgt[0:N_MAIN], axis=0, keepdims=True)
    ap1 = jnp.sum(wgt[ROW_B1:ROW_B1 + NB].reshape(K_SUB, BLOCK, D), axis=1)
    ap2 = jnp.sum(wgt[ROW_B2:ROW_B2 + NB].reshape(K_SUB, BLOCK, D), axis=1)

    def prelu(x, a):
        return jnp.where(x >= 0, x, a * x)

    def mha(q, kv, l):
        lq = q.shape[0]
        qp = _dot(q, wq_ref[l]) + bq_ref[l]
        kp = _dot(kv, wk_ref[l]) + bk_ref[l]
        vp = _dot(kv, wv_ref[l]) + bv_ref[l]
        outs = []
        for hh in range(8):
            sl = slice(16 * hh, 16 * hh + 16)
            att = lax.dot_general(qp[:, sl], kp[:, sl],
                                  (((1,), (1,)), ((), ())),
                                  preferred_element_type=jnp.float32) * 0.25
            att = att - jnp.max(att, axis=-1, keepdims=True)
            att = jnp.exp(att)
            att = att / jnp.sum(att, axis=-1, keepdims=True)
            outs.append(_dot(att, vp[:, sl]))
        o = jnp.concatenate(outs, axis=1)
        return _dot(o, wo_ref[l]) + bo_ref[l]

    def enc(x, kv, l):
        x = _ln(x + mha(x, kv, l), l1s_ref[l], l1b_ref[l])
        f = _dot(jnp.maximum(_dot(x, w1_ref[l]) + b1_ref[l], 0.0), w2_ref[l])
        return _ln(x + f + b2_ref[l], l2s_ref[l], l2b_ref[l])

    def branch(ap, g, lbase):
        fw = fuw_ref[g]
        ap = prelu(_dot(ap, fw[:D]) + _dot(me, fw[D:]) + fub_ref[g],
                   fua_ref[g])
        ap = enc(ap, ap, lbase)
        ap = enc(ap, ap, lbase + 1)
        q = enc(me, ap, lbase + 2)
        q = enc(q, ap, lbase + 3)
        return q

    c1 = branch(ap1, 0, 0)
    c2 = branch(ap2, 1, 4)
    ci = jnp.concatenate([me, c1, c2], axis=1)
    z = prelu(_dot(ci, cw1_ref[:]) + cb1_ref[:], ca_ref[0])
    z = _dot(z, cw2_ref[:]) + cb2_ref[:]
    o_ref[:] = 1.0 / (1.0 + jnp.exp(-z))


def _tail(h, fcw, aw, fu, clf):
    ins = [h, fcw] + aw + fu + clf
    return pl.pallas_call(
        _tail_body,
        out_shape=jax.ShapeDtypeStruct((1, 256), jnp.float32),
    )(*ins)


def kernel(params, x_main, ei_main, ea_main, fcw_main, batch_main,
           x1, ei1, ea1, fcw1, batch1, x2, ei2, ea2, fcw2, batch2):
    gnn = params['gnn']
    f32 = jnp.float32

    zpad_m = jnp.zeros((NM_P - N_MAIN, D), f32)
    x_all = jnp.concatenate([x_main, zpad_m, x1, x2], axis=0)
    fcw_all = jnp.concatenate([
        fcw_main, jnp.zeros((NM_P - N_MAIN,), f32), fcw1, fcw2])[:, None]

    ea_all = jnp.concatenate([
        ea_main, ea1[:BLOCK], ea2[:BLOCK],
        jnp.zeros((E_MAIN_P - E_MAIN - 2 * BLOCK, ea_main.shape[1]), f32)],
        axis=0)

    pad_e = E_MAIN_P - E_MAIN
    src_g = jnp.concatenate([
        ei_main[0], jnp.zeros((pad_e,), jnp.int32),
        ei1[0] + ROW_B1, ei2[0] + ROW_B2])
    dst_l = jnp.concatenate([
        ei_main[1], jnp.full((pad_e,), N_MAIN, jnp.int32),
        ei1[1], ei2[1]])
    sd_pack = jnp.stack(
        [src_g.reshape(-1, _CH), dst_l.reshape(-1, _CH)], axis=1)

    h = _mm_relu(x_all, gnn['W0'], gnn['b0'][None])

    for lp in gnn['layers']:
        wm, bm = lp['Wm'], lp['bm']
        q_mat = _q_proj(ea_all, wm[D:], bm[None])
        qtab = jnp.stack([
            jnp.zeros((BLOCK, D), f32),
            q_mat[E_MAIN:E_MAIN + BLOCK],
            q_mat[E_MAIN + BLOCK:E_MAIN + 2 * BLOCK]])
        p_mat = _p_proj(h, wm[:D], qtab)
        aggp = _edge_sc(p_mat, q_mat, sd_pack)
        h = _update(h, aggp, lp['Wu'][:D], lp['Wu'][D:], lp['bu'][None])

    layers = (params['sa1'] + params['cr1'] + params['sa2'] + params['cr2'])

    def stack(name):
        return jnp.stack([l[name] for l in layers])

    aw = [stack('Wq'), stack('Wk'), stack('Wv'), stack('Wo'),
          stack('bq'), stack('bk'), stack('bv'), stack('bo'),
          stack('ln1_s'), stack('ln1_b'), stack('ln2_s'), stack('ln2_b'),
          stack('W1'), stack('b1'), stack('W2'), stack('b2')]
    fu = [jnp.stack([params['fusion']['W'], params['fusion2']['W']]),
          jnp.stack([params['fusion']['b'], params['fusion2']['b']]),
          jnp.stack([params['fusion']['a'], params['fusion2']['a']])[:, None]]
    cp = params['clf']
    clf = [cp['W1'], cp['b1'][None], jnp.reshape(cp['a'], (1, 1)),
           cp['W2'], cp['b2'][None]]

    return _tail(h, fcw_all, aw, fu, clf)


# speedup vs baseline: 3.4064x; 1.0458x over previous
import functools

import jax
import jax.numpy as jnp
from jax import lax
from jax.experimental import pallas as pl
from jax.experimental.pallas import tpu as pltpu
from jax.experimental.pallas import tpu_sc as plsc

D = 128
N_MAIN = 10000
NM_P = 10240
NB = 8192
ROW_B1 = NM_P
ROW_B2 = NM_P + NB
N_H = NM_P + 2 * NB
E_MAIN = 160000
E_MAIN_P = 163840
EB = 65536
E_ALL = E_MAIN_P + 2 * EB
BLOCK = 512
K_SUB = 16

_SC_NC = 2
_SC_NS = 16

_NAGG = 10048

_PHASES = (
    (0, E_MAIN_P, (100, 60), _NAGG, 0, True),
    (E_MAIN_P, EB, (40, 24), NB, ROW_B1, False),
    (E_MAIN_P + EB, EB, (40, 24), NB, ROW_B2, False),
)
_NSP = _NAGG


_CH = 64


def _edge_sc(p_mat, q_mat, sd_pack):
    mesh = plsc.VectorSubcoreMesh(
        core_axis_name="c", subcore_axis_name="s",
        num_cores=_SC_NC, num_subcores=_SC_NS)

    @functools.partial(
        pl.kernel,
        out_type=jax.ShapeDtypeStruct((_SC_NC, N_H, D), jnp.float32),
        mesh=mesh,
        scratch_types=[
            pltpu.VMEM((2, _CH), jnp.int32),
            pltpu.VMEM((2, _CH), jnp.int32),
            pltpu.VMEM((_CH,), jnp.int32),
            pltpu.VMEM((_CH,), jnp.int32),
            pltpu.VMEM((_CH, D), jnp.float32),
            pltpu.VMEM((_CH, D), jnp.float32),
            pltpu.VMEM((_CH, D), jnp.float32),
            pltpu.VMEM((_CH, D), jnp.float32),
            pltpu.VMEM((_CH, D), jnp.float32),
            pltpu.VMEM((_CH, D), jnp.float32),
            pltpu.VMEM_SHARED((_NSP, D), jnp.float32),
            pltpu.SemaphoreType.DMA,
            pltpu.SemaphoreType.DMA,
            pltpu.SemaphoreType.DMA,
            pltpu.SemaphoreType.DMA,
            pltpu.SemaphoreType.DMA,
            pltpu.SemaphoreType.DMA,
            pltpu.SemaphoreType.DMA,
            pltpu.SemaphoreType.DMA,
        ],
    )
    def k(p_hbm, q_hbm, sd_hbm, out_hbm,
          sd0, sd1, dc0, dc1, mr0, mr1, qr0, qr1, sb0, sb1, agg,
          is0, is1, qs0, qs1, gs0, gs1, ss0, ss1):
        c = lax.axis_index("c")
        s = lax.axis_index("s")
        w = s * _SC_NC + c
        sd = (sd0, sd1)
        dc = (dc0, dc1)
        mr = (mr0, mr1)
        qr = (qr0, qr1)
        sb = (sb0, sb1)
        isem = (is0, is1)
        qsem = (qs0, qs1)
        gsem = (gs0, gs1)
        ssem = (ss0, ss1)

        def drain(sem, dst_ref, dummy_src):
            pltpu.make_async_copy(dummy_src, dst_ref, sem).wait()

        def zero_qr0():
            def zb(r, carry):
                for j in range(D // 16):
                    qr0[r, pl.ds(16 * j, 16)] = jnp.zeros((16,), jnp.float32)
                return carry
            lax.fori_loop(0, _CH, zb, 0)

        first = True
        for (e0, cpe, (cpw0, cpw1), n_loc, row_base, has_q) in _PHASES:
            assert 16 * (cpw0 + cpw1) * _CH == cpe
            cpw = jnp.where(c == 0, cpw0, cpw1)
            cbase = jnp.where(c == 0, s * cpw0, 16 * cpw0 + s * cpw1)
            nch = n_loc // _CH
            ch0 = e0 // _CH

            if first or not has_q:
                zero_qr0()
            first = False

            def zc(i, carry):
                cid = s + i * _SC_NS
                pltpu.sync_copy(qr0, agg.at[pl.ds(cid * _CH, _CH)])
                return carry
            lax.fori_loop(0, (nch - s + _SC_NS - 1) // _SC_NS, zc, 0)
            plsc.subcore_barrier()

            def fire_idx(kk, b):
                cid = ch0 + cbase + kk
                pltpu.async_copy(sd_hbm.at[cid], sd[b], isem[b])

            def fire_q(kk, b):
                cid = ch0 + cbase + kk
                pltpu.async_copy(q_hbm.at[pl.ds(cid * _CH, _CH)],
                                 qr[b], qsem[b])

            def fire_gather(b):
                drain(isem[b], sd[b], sd_hbm.at[0])
                pltpu.async_copy(p_hbm.at[sd[b].at[0]], mr[b], gsem[b])

            def step(kk, b, drain_ssem, prefetch):
                drain(gsem[b], mr[b], p_hbm.at[pl.ds(0, _CH)])
                if drain_ssem:
                    drain(ssem[b], sb[b], p_hbm.at[pl.ds(0, _CH)])
                for j in range(_CH // 16):
                    dc[b][pl.ds(16 * j, 16)] = sd[b][1, pl.ds(16 * j, 16)]
                if prefetch:
                    fire_idx(kk + 2, b)
                if has_q:
                    drain(qsem[b], qr[b], p_hbm.at[pl.ds(0, _CH)])

                def cb(r, carry):
                    for u in range(4):
                        for j in range(D // 16):
                            sl = pl.ds(16 * j, 16)
                            v = mr[b][4 * r + u, sl]
                            if has_q:
                                v = v + qr[b][4 * r + u, sl]
                            sb[b][4 * r + u, sl] = jnp.maximum(v, 0.0)
                    return carry
                lax.fori_loop(0, _CH // 4, cb, 0)

                pltpu.async_copy(sb[b], agg.at[dc[b]], ssem[b], add=True)
                if prefetch:
                    if has_q:
                        fire_q(kk + 2, b)
                    fire_gather(b)

            fire_idx(0, 0)
            fire_idx(1, 1)
            if has_q:
                fire_q(0, 0)
                fire_q(1, 1)
            fire_gather(0)
            fire_gather(1)
            step(0, 0, False, True)
            step(1, 1, False, True)

            def ms(i, carry):
                step(2 + 2 * i, 0, True, True)
                step(3 + 2 * i, 1, True, True)
                return carry
            lax.fori_loop(0, (cpw - 4) // 2, ms, 0)

            step(cpw - 2, 0, True, False)
            step(cpw - 1, 1, True, False)
            drain(ssem[0], sb[0], p_hbm.at[pl.ds(0, _CH)])
            drain(ssem[1], sb[1], p_hbm.at[pl.ds(0, _CH)])
            plsc.subcore_barrier()

            def co(i, carry):
                cid = s + i * _SC_NS
                pltpu.sync_copy(
                    agg.at[pl.ds(cid * _CH, _CH)],
                    out_hbm.at[c, pl.ds(row_base + cid * _CH, _CH)])
                return carry
            lax.fori_loop(0, (nch - s + _SC_NS - 1) // _SC_NS, co, 0)
            plsc.subcore_barrier()

    return k(p_mat, q_mat, sd_pack)


def _mm_relu_body(x_ref, w_ref, b_ref, o_ref):
    o_ref[:] = jnp.maximum(
        jnp.dot(x_ref[:], w_ref[:], preferred_element_type=jnp.float32)
        + b_ref[:], 0.0)


def _mm_relu(x, w, b):
    m = x.shape[0]
    return pl.pallas_call(
        _mm_relu_body,
        out_shape=jax.ShapeDtypeStruct((m, D), jnp.float32),
        grid=(m // 512,),
        in_specs=[
            pl.BlockSpec((512, x.shape[1]), lambda i: (i, 0)),
            pl.BlockSpec((x.shape[1], D), lambda i: (0, 0)),
            pl.BlockSpec((1, D), lambda i: (0, 0)),
        ],
        out_specs=pl.BlockSpec((512, D), lambda i: (i, 0)),
    )(x, w, b)


def _mm_bias_body(x_ref, w_ref, b_ref, o_ref):
    o_ref[:] = (jnp.dot(x_ref[:], w_ref[:], preferred_element_type=jnp.float32)
                + b_ref[:])


def _q_proj(ea, w, b):
    m = ea.shape[0]
    return pl.pallas_call(
        _mm_bias_body,
        out_shape=jax.ShapeDtypeStruct((m, D), jnp.float32),
        grid=(m // 2048,),
        in_specs=[
            pl.BlockSpec((2048, ea.shape[1]), lambda i: (i, 0)),
            pl.BlockSpec((ea.shape[1], D), lambda i: (0, 0)),
            pl.BlockSpec((1, D), lambda i: (0, 0)),
        ],
        out_specs=pl.BlockSpec((2048, D), lambda i: (i, 0)),
    )(ea, w, b)


def _p_body(h_ref, w_ref, qt_ref, o_ref):
    o_ref[:] = (jnp.dot(h_ref[:], w_ref[:], preferred_element_type=jnp.float32)
                + qt_ref[0])


def _qtab_sel(i):
    b1 = ROW_B1 // 512
    b2 = ROW_B2 // 512
    return jnp.where(i < b1, 0, jnp.where(i < b2, 1, 2))


def _p_proj(h, w, qtab):
    return pl.pallas_call(
        _p_body,
        out_shape=jax.ShapeDtypeStruct((N_H, D), jnp.float32),
        grid=(N_H // 512,),
        in_specs=[
            pl.BlockSpec((512, D), lambda i: (i, 0)),
            pl.BlockSpec((D, D), lambda i: (0, 0)),
            pl.BlockSpec((1, 512, D), lambda i: (_qtab_sel(i), 0, 0)),
        ],
        out_specs=pl.BlockSpec((512, D), lambda i: (i, 0)),
    )(h, w, qtab)


def _upd_body(h_ref, ap_ref, wh_ref, wa_ref, b_ref, o_ref):
    a = ap_ref[0] + ap_ref[1]
    o_ref[:] = jnp.maximum(
        jnp.dot(h_ref[:], wh_ref[:], preferred_element_type=jnp.float32)
        + jnp.dot(a, wa_ref[:], preferred_element_type=jnp.float32)
        + b_ref[:], 0.0)


def _update(h, aggp, wh, wa, b):
    return pl.pallas_call(
        _upd_body,
        out_shape=jax.ShapeDtypeStruct((N_H, D), jnp.float32),
        grid=(N_H // 512,),
        in_specs=[
            pl.BlockSpec((512, D), lambda i: (i, 0)),
            pl.BlockSpec((2, 512, D), lambda i: (0, i, 0)),
            pl.BlockSpec((D, D), lambda i: (0, 0)),
            pl.BlockSpec((D, D), lambda i: (0, 0)),
            pl.BlockSpec((1, D), lambda i: (0, 0)),
        ],
        out_specs=pl.BlockSpec((512, D), lambda i: (i, 0)),
    )(h, aggp, wh, wa, b)


def _ln(x, s, b):
    m = jnp.mean(x, axis=-1, keepdims=True)
    v = jnp.mean((x - m) ** 2, axis=-1, keepdims=True)
    return (x - m) * lax.rsqrt(v + 1e-5) * s + b


def _dot(a, b):
    return jnp.dot(a, b, preferred_element_type=jnp.float32)


def _tail_body(h_ref, fcw_ref,
               wq_ref, wk_ref, wv_ref, wo_ref,
               bq_ref, bk_ref, bv_ref, bo_ref,
               l1s_ref, l1b_ref, l2s_ref, l2b_ref,
               w1_ref, b1_ref, w2_ref, b2_ref,
               fuw_ref, fub_ref, fua_ref,
               cw1_ref, cb1_ref, ca_ref, cw2_ref, cb2_ref,
               o_ref):
    wgt = h_ref[:] * fcw_ref[:]
    me = jnp.sum(wgt[0:N_MAIN], axis=0, keepdims=True)
    ap1 = jnp.sum(wgt[ROW_B1:ROW_B1 + NB].reshape(K_SUB, BLOCK, D), axis=1)
    ap2 = jnp.sum(wgt[ROW_B2:ROW_B2 + NB].reshape(K_SUB, BLOCK, D), axis=1)

    def prelu(x, a):
        return jnp.where(x >= 0, x, a * x)

    def mha(q, kv, l):
        lq = q.shape[0]
        qp = _dot(q, wq_ref[l]) + bq_ref[l]
        kp = _dot(kv, wk_ref[l]) + bk_ref[l]
        vp = _dot(kv, wv_ref[l]) + bv_ref[l]
        outs = []
        for hh in range(8):
            sl = slice(16 * hh, 16 * hh + 16)
            att = lax.dot_general(qp[:, sl], kp[:, sl],
                                  (((1,), (1,)), ((), ())),
                                  preferred_element_type=jnp.float32) * 0.25
            att = att - jnp.max(att, axis=-1, keepdims=True)
            att = jnp.exp(att)
            att = att / jnp.sum(att, axis=-1, keepdims=True)
            outs.append(_dot(att, vp[:, sl]))
        o = jnp.concatenate(outs, axis=1)
        return _dot(o, wo_ref[l]) + bo_ref[l]

    def enc(x, kv, l):
        x = _ln(x + mha(x, kv, l), l1s_ref[l], l1b_ref[l])
        f = _dot(jnp.maximum(_dot(x, w1_ref[l]) + b1_ref[l], 0.0), w2_ref[l])
        return _ln(x + f + b2_ref[l], l2s_ref[l], l2b_ref[l])

    def branch(ap, g, lbase):
        fw = fuw_ref[g]
        ap = prelu(_dot(ap, fw[:D]) + _dot(me, fw[D:]) + fub_ref[g],
                   fua_ref[g])
        ap = enc(ap, ap, lbase)
        ap = enc(ap, ap, lbase + 1)
        q = enc(me, ap, lbase + 2)
        q = enc(q, ap, lbase + 3)
        return q

    c1 = branch(ap1, 0, 0)
    c2 = branch(ap2, 1, 4)
    ci = jnp.concatenate([me, c1, c2], axis=1)
    z = prelu(_dot(ci, cw1_ref[:]) + cb1_ref[:], ca_ref[0])
    z = _dot(z, cw2_ref[:]) + cb2_ref[:]
    o_ref[:] = 1.0 / (1.0 + jnp.exp(-z))


def _tail(h, fcw, aw, fu, clf):
    ins = [h, fcw] + aw + fu + clf
    return pl.pallas_call(
        _tail_body,
        out_shape=jax.ShapeDtypeStruct((1, 256), jnp.float32),
    )(*ins)


def kernel(params, x_main, ei_main, ea_main, fcw_main, batch_main,
           x1, ei1, ea1, fcw1, batch1, x2, ei2, ea2, fcw2, batch2):
    gnn = params['gnn']
    f32 = jnp.float32

    zpad_m = jnp.zeros((NM_P - N_MAIN, D), f32)
    x_all = jnp.concatenate([x_main, zpad_m, x1, x2], axis=0)
    fcw_all = jnp.concatenate([
        fcw_main, jnp.zeros((NM_P - N_MAIN,), f32), fcw1, fcw2])[:, None]

    ea_all = jnp.concatenate([
        ea_main, ea1[:BLOCK], ea2[:BLOCK],
        jnp.zeros((E_MAIN_P - E_MAIN - 2 * BLOCK, ea_main.shape[1]), f32)],
        axis=0)

    pad_e = E_MAIN_P - E_MAIN
    src_g = jnp.concatenate([
        ei_main[0], jnp.zeros((pad_e,), jnp.int32),
        ei1[0] + ROW_B1, ei2[0] + ROW_B2])
    dst_l = jnp.concatenate([
        ei_main[1], jnp.full((pad_e,), N_MAIN, jnp.int32),
        ei1[1], ei2[1]])
    sd_pack = jnp.stack(
        [src_g.reshape(-1, _CH), dst_l.reshape(-1, _CH)], axis=1)

    h = _mm_relu(x_all, gnn['W0'], gnn['b0'][None])

    for lp in gnn['layers']:
        wm, bm = lp['Wm'], lp['bm']
        q_mat = _q_proj(ea_all, wm[D:], bm[None])
        qtab = jnp.stack([
            jnp.zeros((BLOCK, D), f32),
            q_mat[E_MAIN:E_MAIN + BLOCK],
            q_mat[E_MAIN + BLOCK:E_MAIN + 2 * BLOCK]])
        p_mat = _p_proj(h, wm[:D], qtab)
        aggp = _edge_sc(p_mat, q_mat, sd_pack)
        h = _update(h, aggp, lp['Wu'][:D], lp['Wu'][D:], lp['bu'][None])

    layers = (params['sa1'] + params['cr1'] + params['sa2'] + params['cr2'])

    def stack(name):
        return jnp.stack([l[name] for l in layers])

    aw = [stack('Wq'), stack('Wk'), stack('Wv'), stack('Wo'),
          stack('bq'), stack('bk'), stack('bv'), stack('bo'),
          stack('ln1_s'), stack('ln1_b'), stack('ln2_s'), stack('ln2_b'),
          stack('W1'), stack('b1'), stack('W2'), stack('b2')]
    fu = [jnp.stack([params['fusion']['W'], params['fusion2']['W']]),
          jnp.stack([params['fusion']['b'], params['fusion2']['b']]),
          jnp.stack([params['fusion']['a'], params['fusion2']['a']])[:, None]]
    cp = params['clf']
    clf = [cp['W1'], cp['b1'][None], jnp.reshape(cp['a'], (1, 1)),
           cp['W2'], cp['b2'][None]]

    return _tail(h, fcw_all, aw, fu, clf)


# speedup vs baseline: 3.4966x; 1.0265x over previous
import functools

import jax
import jax.numpy as jnp
from jax import lax
from jax.experimental import pallas as pl
from jax.experimental.pallas import tpu as pltpu
from jax.experimental.pallas import tpu_sc as plsc

D = 128
N_MAIN = 10000
NM_P = 10240
NB = 8192
ROW_B1 = NM_P
ROW_B2 = NM_P + NB
N_H = NM_P + 2 * NB
E_MAIN = 160000
E_MAIN_P = 163840
EB = 65536
E_ALL = E_MAIN_P + 2 * EB
BLOCK = 512
K_SUB = 16

_SC_NC = 2
_SC_NS = 16

_NAGG = 10048

_PHASES = (
    (0, E_MAIN_P, (100, 60), _NAGG, 0, True),
    (E_MAIN_P, EB, (40, 24), NB, ROW_B1, False),
    (E_MAIN_P + EB, EB, (40, 24), NB, ROW_B2, False),
)
_NSP = _NAGG


_CH = 64


def _edge_sc(p_mat, q_mat, sd_pack):
    mesh = plsc.VectorSubcoreMesh(
        core_axis_name="c", subcore_axis_name="s",
        num_cores=_SC_NC, num_subcores=_SC_NS)

    @functools.partial(
        pl.kernel,
        out_type=jax.ShapeDtypeStruct((_SC_NC, N_H, D), jnp.float32),
        mesh=mesh,
        scratch_types=[
            pltpu.VMEM((2, _CH), jnp.int32),
            pltpu.VMEM((2, _CH), jnp.int32),
            pltpu.VMEM((_CH,), jnp.int32),
            pltpu.VMEM((_CH,), jnp.int32),
            pltpu.VMEM((_CH, D), jnp.float32),
            pltpu.VMEM((_CH, D), jnp.float32),
            pltpu.VMEM((_CH, D), jnp.float32),
            pltpu.VMEM((_CH, D), jnp.float32),
            pltpu.VMEM((_CH, D), jnp.float32),
            pltpu.VMEM((_CH, D), jnp.float32),
            pltpu.VMEM_SHARED((_NSP, D), jnp.float32),
            pltpu.SemaphoreType.DMA,
            pltpu.SemaphoreType.DMA,
            pltpu.SemaphoreType.DMA,
            pltpu.SemaphoreType.DMA,
            pltpu.SemaphoreType.DMA,
            pltpu.SemaphoreType.DMA,
            pltpu.SemaphoreType.DMA,
            pltpu.SemaphoreType.DMA,
        ],
    )
    def k(p_hbm, q_hbm, sd_hbm, out_hbm,
          sd0, sd1, dc0, dc1, mr0, mr1, qr0, qr1, sb0, sb1, agg,
          is0, is1, qs0, qs1, gs0, gs1, ss0, ss1):
        c = lax.axis_index("c")
        s = lax.axis_index("s")
        w = s * _SC_NC + c
        sd = (sd0, sd1)
        dc = (dc0, dc1)
        mr = (mr0, mr1)
        qr = (qr0, qr1)
        sb = (sb0, sb1)
        isem = (is0, is1)
        qsem = (qs0, qs1)
        gsem = (gs0, gs1)
        ssem = (ss0, ss1)

        def drain(sem, dst_ref, dummy_src):
            pltpu.make_async_copy(dummy_src, dst_ref, sem).wait()

        def zero_qr0():
            def zb(r, carry):
                for j in range(D // 16):
                    qr0[r, pl.ds(16 * j, 16)] = jnp.zeros((16,), jnp.float32)
                return carry
            lax.fori_loop(0, _CH, zb, 0)

        first = True
        for (e0, cpe, (cpw0, cpw1), n_loc, row_base, has_q) in _PHASES:
            assert 16 * (cpw0 + cpw1) * _CH == cpe
            cpw = jnp.where(c == 0, cpw0, cpw1)
            cbase = jnp.where(c == 0, s * cpw0, 16 * cpw0 + s * cpw1)
            nch = n_loc // _CH
            ch0 = e0 // _CH

            if first or not has_q:
                zero_qr0()
            first = False

            nz = (nch - s + _SC_NS - 1) // _SC_NS

            def zc(i, carry):
                cid = s + i * _SC_NS
                pltpu.sync_copy(qr0, agg.at[pl.ds(cid * _CH, _CH)])
                return carry
            lax.fori_loop(0, nz, zc, 0)
            plsc.subcore_barrier()

            def fire_idx(kk, b):
                cid = ch0 + cbase + kk
                pltpu.async_copy(sd_hbm.at[cid], sd[b], isem[b])

            def fire_q(kk, b):
                cid = ch0 + cbase + kk
                pltpu.async_copy(q_hbm.at[pl.ds(cid * _CH, _CH)],
                                 qr[b], qsem[b])

            def fire_gather(b):
                drain(isem[b], sd[b], sd_hbm.at[0])
                pltpu.async_copy(p_hbm.at[sd[b].at[0]], mr[b], gsem[b])

            def step(kk, b, drain_ssem, prefetch):
                drain(gsem[b], mr[b], p_hbm.at[pl.ds(0, _CH)])
                if drain_ssem:
                    drain(ssem[b], sb[b], p_hbm.at[pl.ds(0, _CH)])
                for j in range(_CH // 16):
                    dc[b][pl.ds(16 * j, 16)] = sd[b][1, pl.ds(16 * j, 16)]
                if prefetch:
                    fire_idx(kk + 2, b)
                if has_q:
                    drain(qsem[b], qr[b], p_hbm.at[pl.ds(0, _CH)])

                def cb(r, carry):
                    for u in range(4):
                        for j in range(D // 16):
                            sl = pl.ds(16 * j, 16)
                            v = mr[b][4 * r + u, sl]
                            if has_q:
                                v = v + qr[b][4 * r + u, sl]
                            sb[b][4 * r + u, sl] = jnp.maximum(v, 0.0)
                    return carry
                lax.fori_loop(0, _CH // 4, cb, 0)

                pltpu.async_copy(sb[b], agg.at[dc[b]], ssem[b], add=True)
                if prefetch:
                    if has_q:
                        fire_q(kk + 2, b)
                    fire_gather(b)

            fire_idx(0, 0)
            fire_idx(1, 1)
            if has_q:
                fire_q(0, 0)
                fire_q(1, 1)
            fire_gather(0)
            fire_gather(1)
            step(0, 0, False, True)
            step(1, 1, False, True)

            def ms(i, carry):
                step(2 + 2 * i, 0, True, True)
                step(3 + 2 * i, 1, True, True)
                return carry
            lax.fori_loop(0, (cpw - 4) // 2, ms, 0)

            step(cpw - 2, 0, True, False)
            step(cpw - 1, 1, True, False)
            drain(ssem[0], sb[0], p_hbm.at[pl.ds(0, _CH)])
            drain(ssem[1], sb[1], p_hbm.at[pl.ds(0, _CH)])
            plsc.subcore_barrier()

            def co(i, carry):
                cid = s + i * _SC_NS
                pltpu.sync_copy(
                    agg.at[pl.ds(cid * _CH, _CH)],
                    out_hbm.at[c, pl.ds(row_base + cid * _CH, _CH)])
                return carry
            lax.fori_loop(0, nz, co, 0)
            plsc.subcore_barrier()

    return k(p_mat, q_mat, sd_pack)


def _mm_relu_body(x_ref, w_ref, b_ref, o_ref):
    o_ref[:] = jnp.maximum(
        jnp.dot(x_ref[:], w_ref[:], preferred_element_type=jnp.float32)
        + b_ref[:], 0.0)


def _mm_relu(x, w, b):
    m = x.shape[0]
    return pl.pallas_call(
        _mm_relu_body,
        out_shape=jax.ShapeDtypeStruct((m, D), jnp.float32),
        grid=(m // 512,),
        in_specs=[
            pl.BlockSpec((512, x.shape[1]), lambda i: (i, 0)),
            pl.BlockSpec((x.shape[1], D), lambda i: (0, 0)),
            pl.BlockSpec((1, D), lambda i: (0, 0)),
        ],
        out_specs=pl.BlockSpec((512, D), lambda i: (i, 0)),
    )(x, w, b)


def _mm_bias_body(x_ref, w_ref, b_ref, o_ref):
    o_ref[:] = (jnp.dot(x_ref[:], w_ref[:], preferred_element_type=jnp.float32)
                + b_ref[:])


def _mm_bias_dep_body(x_ref, w_ref, b_ref, d_ref, o_ref):
    del d_ref
    o_ref[:] = (jnp.dot(x_ref[:], w_ref[:], preferred_element_type=jnp.float32)
                + b_ref[:])


def _q_proj(ea, w, b, dep=None):
    m = ea.shape[0]
    ins = [ea, w, b]
    body = _mm_bias_body
    in_specs = [
        pl.BlockSpec((2048, ea.shape[1]), lambda i: (i, 0)),
        pl.BlockSpec((ea.shape[1], D), lambda i: (0, 0)),
        pl.BlockSpec((1, D), lambda i: (0, 0)),
    ]
    if dep is not None:
        ins.append(dep)
        body = _mm_bias_dep_body
        in_specs.append(pl.BlockSpec((1, D), lambda i: (0, 0)))
    return pl.pallas_call(
        body,
        out_shape=jax.ShapeDtypeStruct((m, D), jnp.float32),
        grid=(m // 2048,),
        in_specs=in_specs,
        out_specs=pl.BlockSpec((2048, D), lambda i: (i, 0)),
    )(*ins)


def _p_body(h_ref, w_ref, qt_ref, o_ref):
    o_ref[:] = (jnp.dot(h_ref[:], w_ref[:], preferred_element_type=jnp.float32)
                + qt_ref[0])


def _qtab_sel(i):
    b1 = ROW_B1 // 512
    b2 = ROW_B2 // 512
    return jnp.where(i < b1, 0, jnp.where(i < b2, 1, 2))


def _p_proj(h, w, qtab):
    return pl.pallas_call(
        _p_body,
        out_shape=jax.ShapeDtypeStruct((N_H, D), jnp.float32),
        grid=(N_H // 512,),
        in_specs=[
            pl.BlockSpec((512, D), lambda i: (i, 0)),
            pl.BlockSpec((D, D), lambda i: (0, 0)),
            pl.BlockSpec((1, 512, D), lambda i: (_qtab_sel(i), 0, 0)),
        ],
        out_specs=pl.BlockSpec((512, D), lambda i: (i, 0)),
    )(h, w, qtab)


def _upd_body(h_ref, ap_ref, wh_ref, wa_ref, b_ref, o_ref):
    a = ap_ref[0] + ap_ref[1]
    o_ref[:] = jnp.maximum(
        jnp.dot(h_ref[:], wh_ref[:], preferred_element_type=jnp.float32)
        + jnp.dot(a, wa_ref[:], preferred_element_type=jnp.float32)
        + b_ref[:], 0.0)


def _update(h, aggp, wh, wa, b):
    return pl.pallas_call(
        _upd_body,
        out_shape=jax.ShapeDtypeStruct((N_H, D), jnp.float32),
        grid=(N_H // 512,),
        in_specs=[
            pl.BlockSpec((512, D), lambda i: (i, 0)),
            pl.BlockSpec((2, 512, D), lambda i: (0, i, 0)),
            pl.BlockSpec((D, D), lambda i: (0, 0)),
            pl.BlockSpec((D, D), lambda i: (0, 0)),
            pl.BlockSpec((1, D), lambda i: (0, 0)),
        ],
        out_specs=pl.BlockSpec((512, D), lambda i: (i, 0)),
    )(h, aggp, wh, wa, b)


def _ln(x, s, b):
    m = jnp.mean(x, axis=-1, keepdims=True)
    v = jnp.mean((x - m) ** 2, axis=-1, keepdims=True)
    return (x - m) * lax.rsqrt(v + 1e-5) * s + b


def _dot(a, b):
    return jnp.dot(a, b, preferred_element_type=jnp.float32)


def _tail_body(h_ref, fcw_ref,
               wq_ref, wk_ref, wv_ref, wo_ref,
               bq_ref, bk_ref, bv_ref, bo_ref,
               l1s_ref, l1b_ref, l2s_ref, l2b_ref,
               w1_ref, b1_ref, w2_ref, b2_ref,
               fuw_ref, fub_ref, fua_ref,
               cw1_ref, cb1_ref, ca_ref, cw2_ref, cb2_ref,
               o_ref):
    wgt = h_ref[:] * fcw_ref[:]
    me = jnp.sum(wgt[0:N_MAIN], axis=0, keepdims=True)
    ap1 = jnp.sum(wgt[ROW_B1:ROW_B1 + NB].reshape(K_SUB, BLOCK, D), axis=1)
    ap2 = jnp.sum(wgt[ROW_B2:ROW_B2 + NB].reshape(K_SUB, BLOCK, D), axis=1)

    def prelu(x, a):
        return jnp.where(x >= 0, x, a * x)

    def mha(q, kv, l):
        lq = q.shape[0]
        qp = _dot(q, wq_ref[l]) + bq_ref[l]
        kp = _dot(kv, wk_ref[l]) + bk_ref[l]
        vp = _dot(kv, wv_ref[l]) + bv_ref[l]
        outs = []
        for hh in range(8):
            sl = slice(16 * hh, 16 * hh + 16)
            att = lax.dot_general(qp[:, sl], kp[:, sl],
                                  (((1,), (1,)), ((), ())),
                                  preferred_element_type=jnp.float32) * 0.25
            att = att - jnp.max(att, axis=-1, keepdims=True)
            att = jnp.exp(att)
            att = att / jnp.sum(att, axis=-1, keepdims=True)
            outs.append(_dot(att, vp[:, sl]))
        o = jnp.concatenate(outs, axis=1)
        return _dot(o, wo_ref[l]) + bo_ref[l]

    def enc(x, kv, l):
        x = _ln(x + mha(x, kv, l), l1s_ref[l], l1b_ref[l])
        f = _dot(jnp.maximum(_dot(x, w1_ref[l]) + b1_ref[l], 0.0), w2_ref[l])
        return _ln(x + f + b2_ref[l], l2s_ref[l], l2b_ref[l])

    def branch(ap, g, lbase):
        fw = fuw_ref[g]
        ap = prelu(_dot(ap, fw[:D]) + _dot(me, fw[D:]) + fub_ref[g],
                   fua_ref[g])
        ap = enc(ap, ap, lbase)
        ap = enc(ap, ap, lbase + 1)
        q = enc(me, ap, lbase + 2)
        q = enc(q, ap, lbase + 3)
        return q

    c1 = branch(ap1, 0, 0)
    c2 = branch(ap2, 1, 4)
    ci = jnp.concatenate([me, c1, c2], axis=1)
    z = prelu(_dot(ci, cw1_ref[:]) + cb1_ref[:], ca_ref[0])
    z = _dot(z, cw2_ref[:]) + cb2_ref[:]
    o_ref[:] = 1.0 / (1.0 + jnp.exp(-z))


def _tail(h, fcw, aw, fu, clf):
    ins = [h, fcw] + aw + fu + clf
    return pl.pallas_call(
        _tail_body,
        out_shape=jax.ShapeDtypeStruct((1, 256), jnp.float32),
    )(*ins)


def kernel(params, x_main, ei_main, ea_main, fcw_main, batch_main,
           x1, ei1, ea1, fcw1, batch1, x2, ei2, ea2, fcw2, batch2):
    gnn = params['gnn']
    f32 = jnp.float32

    zpad_m = jnp.zeros((NM_P - N_MAIN, D), f32)
    x_all = jnp.concatenate([x_main, zpad_m, x1, x2], axis=0)
    fcw_all = jnp.concatenate([
        fcw_main, jnp.zeros((NM_P - N_MAIN,), f32), fcw1, fcw2])[:, None]

    ea_all = jnp.concatenate([
        ea_main, ea1[:BLOCK], ea2[:BLOCK],
        jnp.zeros((E_MAIN_P - E_MAIN - 2 * BLOCK, ea_main.shape[1]), f32)],
        axis=0)

    pad_e = E_MAIN_P - E_MAIN
    src_g = jnp.concatenate([
        ei_main[0], jnp.zeros((pad_e,), jnp.int32),
        ei1[0] + ROW_B1, ei2[0] + ROW_B2])
    dst_l = jnp.concatenate([
        ei_main[1], jnp.full((pad_e,), N_MAIN, jnp.int32),
        ei1[1], ei2[1]])
    sd_pack = jnp.stack(
        [src_g.reshape(-1, _CH), dst_l.reshape(-1, _CH)], axis=1)

    h = _mm_relu(x_all, gnn['W0'], gnn['b0'][None])

    prev_p = None
    for lp in gnn['layers']:
        wm, bm = lp['Wm'], lp['bm']
        q_mat = _q_proj(ea_all, wm[D:], bm[None],
                        None if prev_p is None else prev_p[:1])
        qtab = jnp.stack([
            jnp.zeros((BLOCK, D), f32),
            q_mat[E_MAIN:E_MAIN + BLOCK],
            q_mat[E_MAIN + BLOCK:E_MAIN + 2 * BLOCK]])
        p_mat = _p_proj(h, wm[:D], qtab)
        aggp = _edge_sc(p_mat, q_mat, sd_pack)
        h = _update(h, aggp, lp['Wu'][:D], lp['Wu'][D:], lp['bu'][None])
        prev_p = p_mat

    layers = (params['sa1'] + params['cr1'] + params['sa2'] + params['cr2'])

    def stack(name):
        return jnp.stack([l[name] for l in layers])

    aw = [stack('Wq'), stack('Wk'), stack('Wv'), stack('Wo'),
          stack('bq'), stack('bk'), stack('bv'), stack('bo'),
          stack('ln1_s'), stack('ln1_b'), stack('ln2_s'), stack('ln2_b'),
          stack('W1'), stack('b1'), stack('W2'), stack('b2')]
    fu = [jnp.stack([params['fusion']['W'], params['fusion2']['W']]),
          jnp.stack([params['fusion']['b'], params['fusion2']['b']]),
          jnp.stack([params['fusion']['a'], params['fusion2']['a']])[:, None]]
    cp = params['clf']
    clf = [cp['W1'], cp['b1'][None], jnp.reshape(cp['a'], (1, 1)),
           cp['W2'], cp['b2'][None]]

    return _tail(h, fcw_all, aw, fu, clf)


# speedup vs baseline: 3.5183x; 1.0062x over previous
import functools

import jax
import jax.numpy as jnp
from jax import lax
from jax.experimental import pallas as pl
from jax.experimental.pallas import tpu as pltpu
from jax.experimental.pallas import tpu_sc as plsc

D = 128
N_MAIN = 10000
NM_P = 10240
NB = 8192
ROW_B1 = NM_P
ROW_B2 = NM_P + NB
N_H = NM_P + 2 * NB
E_MAIN = 160000
E_MAIN_P = 163840
EB = 65536
E_ALL = E_MAIN_P + 2 * EB
BLOCK = 512
K_SUB = 16

_SC_NC = 2
_SC_NS = 16

_NAGG = 10112

_PHASES = (
    (0, E_MAIN_P, (100, 60), _NAGG, 0, True, 128),
    (E_MAIN_P, EB, (40, 24), NB, ROW_B1, False, 256),
    (E_MAIN_P + EB, EB, (40, 24), NB, ROW_B2, False, 256),
)
_NSP = _NAGG


_CH = 64


def _edge_sc(p_mat, q_mat, sd_pack):
    mesh = plsc.VectorSubcoreMesh(
        core_axis_name="c", subcore_axis_name="s",
        num_cores=_SC_NC, num_subcores=_SC_NS)

    @functools.partial(
        pl.kernel,
        out_type=jax.ShapeDtypeStruct((_SC_NC, N_H, D), jnp.float32),
        mesh=mesh,
        scratch_types=[
            pltpu.VMEM((2, _CH), jnp.int32),
            pltpu.VMEM((2, _CH), jnp.int32),
            pltpu.VMEM((_CH,), jnp.int32),
            pltpu.VMEM((_CH,), jnp.int32),
            pltpu.VMEM((_CH, D), jnp.float32),
            pltpu.VMEM((_CH, D), jnp.float32),
            pltpu.VMEM((_CH, D), jnp.float32),
            pltpu.VMEM((_CH, D), jnp.float32),
            pltpu.VMEM((_CH, D), jnp.float32),
            pltpu.VMEM((_CH, D), jnp.float32),
            pltpu.VMEM_SHARED((_NSP, D), jnp.float32),
            pltpu.SemaphoreType.DMA,
            pltpu.SemaphoreType.DMA,
            pltpu.SemaphoreType.DMA,
            pltpu.SemaphoreType.DMA,
            pltpu.SemaphoreType.DMA,
            pltpu.SemaphoreType.DMA,
            pltpu.SemaphoreType.DMA,
            pltpu.SemaphoreType.DMA,
        ],
    )
    def k(p_hbm, q_hbm, sd_hbm, out_hbm,
          sd0, sd1, dc0, dc1, mr0, mr1, qr0, qr1, sb0, sb1, agg,
          is0, is1, qs0, qs1, gs0, gs1, ss0, ss1):
        c = lax.axis_index("c")
        s = lax.axis_index("s")
        w = s * _SC_NC + c
        sd = (sd0, sd1)
        dc = (dc0, dc1)
        mr = (mr0, mr1)
        qr = (qr0, qr1)
        sb = (sb0, sb1)
        isem = (is0, is1)
        qsem = (qs0, qs1)
        gsem = (gs0, gs1)
        ssem = (ss0, ss1)

        def drain(sem, dst_ref, dummy_src):
            pltpu.make_async_copy(dummy_src, dst_ref, sem).wait()

        def zero_qr0():
            def zb(r, carry):
                for j in range(D // 16):
                    qr0[r, pl.ds(16 * j, 16)] = jnp.zeros((16,), jnp.float32)
                return carry
            lax.fori_loop(0, _CH, zb, 0)

        first = True
        for (e0, cpe, (cpw0, cpw1), n_loc, row_base, has_q, dch) in _PHASES:
            assert 16 * (cpw0 + cpw1) * _CH == cpe
            cpw = jnp.where(c == 0, cpw0, cpw1)
            cbase = jnp.where(c == 0, s * cpw0, 16 * cpw0 + s * cpw1)
            nch = n_loc // _CH
            ch0 = e0 // _CH

            if first or not has_q:
                zero_qr0()
            first = False

            nz = (nch - s + _SC_NS - 1) // _SC_NS

            def zc(i, carry):
                cid = s + i * _SC_NS
                pltpu.sync_copy(qr0, agg.at[pl.ds(cid * _CH, _CH)])
                return carry
            lax.fori_loop(0, nz, zc, 0)
            plsc.subcore_barrier()

            def fire_idx(kk, b):
                cid = ch0 + cbase + kk
                pltpu.async_copy(sd_hbm.at[cid], sd[b], isem[b])

            def fire_q(kk, b):
                cid = ch0 + cbase + kk
                pltpu.async_copy(q_hbm.at[pl.ds(cid * _CH, _CH)],
                                 qr[b], qsem[b])

            def fire_gather(b):
                drain(isem[b], sd[b], sd_hbm.at[0])
                pltpu.async_copy(p_hbm.at[sd[b].at[0]], mr[b], gsem[b])

            def step(kk, b, drain_ssem, prefetch):
                drain(gsem[b], mr[b], p_hbm.at[pl.ds(0, _CH)])
                if drain_ssem:
                    drain(ssem[b], sb[b], p_hbm.at[pl.ds(0, _CH)])
                for j in range(_CH // 16):
                    dc[b][pl.ds(16 * j, 16)] = sd[b][1, pl.ds(16 * j, 16)]
                if prefetch:
                    fire_idx(kk + 2, b)
                if has_q:
                    drain(qsem[b], qr[b], p_hbm.at[pl.ds(0, _CH)])

                def cb(r, carry):
                    for u in range(4):
                        for j in range(D // 16):
                            sl = pl.ds(16 * j, 16)
                            v = mr[b][4 * r + u, sl]
                            if has_q:
                                v = v + qr[b][4 * r + u, sl]
                            sb[b][4 * r + u, sl] = jnp.maximum(v, 0.0)
                    return carry
                lax.fori_loop(0, _CH // 4, cb, 0)

                pltpu.async_copy(sb[b], agg.at[dc[b]], ssem[b], add=True)
                if prefetch:
                    if has_q:
                        fire_q(kk + 2, b)
                    fire_gather(b)

            fire_idx(0, 0)
            fire_idx(1, 1)
            if has_q:
                fire_q(0, 0)
                fire_q(1, 1)
            fire_gather(0)
            fire_gather(1)
            step(0, 0, False, True)
            step(1, 1, False, True)

            def ms(i, carry):
                step(2 + 2 * i, 0, True, True)
                step(3 + 2 * i, 1, True, True)
                return carry
            lax.fori_loop(0, (cpw - 4) // 2, ms, 0)

            step(cpw - 2, 0, True, False)
            step(cpw - 1, 1, True, False)
            drain(ssem[0], sb[0], p_hbm.at[pl.ds(0, _CH)])
            drain(ssem[1], sb[1], p_hbm.at[pl.ds(0, _CH)])
            plsc.subcore_barrier()

            nu = n_loc // dch

            def co(i, carry):
                cid = s + i * _SC_NS
                pltpu.sync_copy(
                    agg.at[pl.ds(cid * dch, dch)],
                    out_hbm.at[c, pl.ds(row_base + cid * dch, dch)])
                return carry
            lax.fori_loop(0, (nu - s + _SC_NS - 1) // _SC_NS, co, 0)
            plsc.subcore_barrier()

    return k(p_mat, q_mat, sd_pack)


def _mm_relu_body(x_ref, w_ref, b_ref, o_ref):
    o_ref[:] = jnp.maximum(
        jnp.dot(x_ref[:], w_ref[:], preferred_element_type=jnp.float32)
        + b_ref[:], 0.0)


def _mm_relu(x, w, b):
    m = x.shape[0]
    return pl.pallas_call(
        _mm_relu_body,
        out_shape=jax.ShapeDtypeStruct((m, D), jnp.float32),
        grid=(m // 512,),
        in_specs=[
            pl.BlockSpec((512, x.shape[1]), lambda i: (i, 0)),
            pl.BlockSpec((x.shape[1], D), lambda i: (0, 0)),
            pl.BlockSpec((1, D), lambda i: (0, 0)),
        ],
        out_specs=pl.BlockSpec((512, D), lambda i: (i, 0)),
    )(x, w, b)


def _mm_bias_body(x_ref, w_ref, b_ref, o_ref):
    o_ref[:] = (jnp.dot(x_ref[:], w_ref[:], preferred_element_type=jnp.float32)
                + b_ref[:])


def _mm_bias_dep_body(x_ref, w_ref, b_ref, d_ref, o_ref):
    del d_ref
    o_ref[:] = (jnp.dot(x_ref[:], w_ref[:], preferred_element_type=jnp.float32)
                + b_ref[:])


def _q_proj(ea, w, b, dep=None):
    m = ea.shape[0]
    ins = [ea, w, b]
    body = _mm_bias_body
    in_specs = [
        pl.BlockSpec((2048, ea.shape[1]), lambda i: (i, 0)),
        pl.BlockSpec((ea.shape[1], D), lambda i: (0, 0)),
        pl.BlockSpec((1, D), lambda i: (0, 0)),
    ]
    if dep is not None:
        ins.append(dep)
        body = _mm_bias_dep_body
        in_specs.append(pl.BlockSpec((1, D), lambda i: (0, 0)))
    return pl.pallas_call(
        body,
        out_shape=jax.ShapeDtypeStruct((m, D), jnp.float32),
        grid=(m // 2048,),
        in_specs=in_specs,
        out_specs=pl.BlockSpec((2048, D), lambda i: (i, 0)),
    )(*ins)


def _p_body(h_ref, w_ref, qt_ref, o_ref):
    o_ref[:] = (jnp.dot(h_ref[:], w_ref[:], preferred_element_type=jnp.float32)
                + qt_ref[0])


def _qtab_sel(i):
    b1 = ROW_B1 // 512
    b2 = ROW_B2 // 512
    return jnp.where(i < b1, 0, jnp.where(i < b2, 1, 2))


def _p_proj(h, w, qtab):
    return pl.pallas_call(
        _p_body,
        out_shape=jax.ShapeDtypeStruct((N_H, D), jnp.float32),
        grid=(N_H // 512,),
        in_specs=[
            pl.BlockSpec((512, D), lambda i: (i, 0)),
            pl.BlockSpec((D, D), lambda i: (0, 0)),
            pl.BlockSpec((1, 512, D), lambda i: (_qtab_sel(i), 0, 0)),
        ],
        out_specs=pl.BlockSpec((512, D), lambda i: (i, 0)),
    )(h, w, qtab)


def _upd_body(h_ref, ap_ref, wh_ref, wa_ref, b_ref, o_ref):
    a = ap_ref[0] + ap_ref[1]
    o_ref[:] = jnp.maximum(
        jnp.dot(h_ref[:], wh_ref[:], preferred_element_type=jnp.float32)
        + jnp.dot(a, wa_ref[:], preferred_element_type=jnp.float32)
        + b_ref[:], 0.0)


def _update(h, aggp, wh, wa, b):
    return pl.pallas_call(
        _upd_body,
        out_shape=jax.ShapeDtypeStruct((N_H, D), jnp.float32),
        grid=(N_H // 512,),
        in_specs=[
            pl.BlockSpec((512, D), lambda i: (i, 0)),
            pl.BlockSpec((2, 512, D), lambda i: (0, i, 0)),
            pl.BlockSpec((D, D), lambda i: (0, 0)),
            pl.BlockSpec((D, D), lambda i: (0, 0)),
            pl.BlockSpec((1, D), lambda i: (0, 0)),
        ],
        out_specs=pl.BlockSpec((512, D), lambda i: (i, 0)),
    )(h, aggp, wh, wa, b)


def _ln(x, s, b):
    m = jnp.mean(x, axis=-1, keepdims=True)
    v = jnp.mean((x - m) ** 2, axis=-1, keepdims=True)
    return (x - m) * lax.rsqrt(v + 1e-5) * s + b


def _dot(a, b):
    return jnp.dot(a, b, preferred_element_type=jnp.float32)


def _tail_body(h_ref, fcw_ref,
               wq_ref, wk_ref, wv_ref, wo_ref,
               bq_ref, bk_ref, bv_ref, bo_ref,
               l1s_ref, l1b_ref, l2s_ref, l2b_ref,
               w1_ref, b1_ref, w2_ref, b2_ref,
               fuw_ref, fub_ref, fua_ref,
               cw1_ref, cb1_ref, ca_ref, cw2_ref, cb2_ref,
               o_ref):
    wgt = h_ref[:] * fcw_ref[:]
    me = jnp.sum(wgt[0:N_MAIN], axis=0, keepdims=True)
    ap1 = jnp.sum(wgt[ROW_B1:ROW_B1 + NB].reshape(K_SUB, BLOCK, D), axis=1)
    ap2 = jnp.sum(wgt[ROW_B2:ROW_B2 + NB].reshape(K_SUB, BLOCK, D), axis=1)

    def prelu(x, a):
        return jnp.where(x >= 0, x, a * x)

    def mha(q, kv, l):
        lq = q.shape[0]
        qp = _dot(q, wq_ref[l]) + bq_ref[l]
        kp = _dot(kv, wk_ref[l]) + bk_ref[l]
        vp = _dot(kv, wv_ref[l]) + bv_ref[l]
        outs = []
        for hh in range(8):
            sl = slice(16 * hh, 16 * hh + 16)
            att = lax.dot_general(qp[:, sl], kp[:, sl],
                                  (((1,), (1,)), ((), ())),
                                  preferred_element_type=jnp.float32) * 0.25
            att = att - jnp.max(att, axis=-1, keepdims=True)
            att = jnp.exp(att)
            att = att / jnp.sum(att, axis=-1, keepdims=True)
            outs.append(_dot(att, vp[:, sl]))
        o = jnp.concatenate(outs, axis=1)
        return _dot(o, wo_ref[l]) + bo_ref[l]

    def enc(x, kv, l):
        x = _ln(x + mha(x, kv, l), l1s_ref[l], l1b_ref[l])
        f = _dot(jnp.maximum(_dot(x, w1_ref[l]) + b1_ref[l], 0.0), w2_ref[l])
        return _ln(x + f + b2_ref[l], l2s_ref[l], l2b_ref[l])

    def branch(ap, g, lbase):
        fw = fuw_ref[g]
        ap = prelu(_dot(ap, fw[:D]) + _dot(me, fw[D:]) + fub_ref[g],
                   fua_ref[g])
        ap = enc(ap, ap, lbase)
        ap = enc(ap, ap, lbase + 1)
        q = enc(me, ap, lbase + 2)
        q = enc(q, ap, lbase + 3)
        return q

    c1 = branch(ap1, 0, 0)
    c2 = branch(ap2, 1, 4)
    ci = jnp.concatenate([me, c1, c2], axis=1)
    z = prelu(_dot(ci, cw1_ref[:]) + cb1_ref[:], ca_ref[0])
    z = _dot(z, cw2_ref[:]) + cb2_ref[:]
    o_ref[:] = 1.0 / (1.0 + jnp.exp(-z))


def _tail(h, fcw, aw, fu, clf):
    ins = [h, fcw] + aw + fu + clf
    return pl.pallas_call(
        _tail_body,
        out_shape=jax.ShapeDtypeStruct((1, 256), jnp.float32),
    )(*ins)


def kernel(params, x_main, ei_main, ea_main, fcw_main, batch_main,
           x1, ei1, ea1, fcw1, batch1, x2, ei2, ea2, fcw2, batch2):
    gnn = params['gnn']
    f32 = jnp.float32

    zpad_m = jnp.zeros((NM_P - N_MAIN, D), f32)
    x_all = jnp.concatenate([x_main, zpad_m, x1, x2], axis=0)
    fcw_all = jnp.concatenate([
        fcw_main, jnp.zeros((NM_P - N_MAIN,), f32), fcw1, fcw2])[:, None]

    ea_all = jnp.concatenate([
        ea_main, ea1[:BLOCK], ea2[:BLOCK],
        jnp.zeros((E_MAIN_P - E_MAIN - 2 * BLOCK, ea_main.shape[1]), f32)],
        axis=0)

    pad_e = E_MAIN_P - E_MAIN
    src_g = jnp.concatenate([
        ei_main[0], jnp.zeros((pad_e,), jnp.int32),
        ei1[0] + ROW_B1, ei2[0] + ROW_B2])
    dst_l = jnp.concatenate([
        ei_main[1], jnp.full((pad_e,), N_MAIN, jnp.int32),
        ei1[1], ei2[1]])
    sd_pack = jnp.stack(
        [src_g.reshape(-1, _CH), dst_l.reshape(-1, _CH)], axis=1)

    h = _mm_relu(x_all, gnn['W0'], gnn['b0'][None])

    prev_p = None
    for lp in gnn['layers']:
        wm, bm = lp['Wm'], lp['bm']
        q_mat = _q_proj(ea_all, wm[D:], bm[None],
                        None if prev_p is None else prev_p[:1])
        qtab = jnp.stack([
            jnp.zeros((BLOCK, D), f32),
            q_mat[E_MAIN:E_MAIN + BLOCK],
            q_mat[E_MAIN + BLOCK:E_MAIN + 2 * BLOCK]])
        p_mat = _p_proj(h, wm[:D], qtab)
        aggp = _edge_sc(p_mat, q_mat, sd_pack)
        h = _update(h, aggp, lp['Wu'][:D], lp['Wu'][D:], lp['bu'][None])
        prev_p = p_mat

    layers = (params['sa1'] + params['cr1'] + params['sa2'] + params['cr2'])

    def stack(name):
        return jnp.stack([l[name] for l in layers])

    aw = [stack('Wq'), stack('Wk'), stack('Wv'), stack('Wo'),
          stack('bq'), stack('bk'), stack('bv'), stack('bo'),
          stack('ln1_s'), stack('ln1_b'), stack('ln2_s'), stack('ln2_b'),
          stack('W1'), stack('b1'), stack('W2'), stack('b2')]
    fu = [jnp.stack([params['fusion']['W'], params['fusion2']['W']]),
          jnp.stack([params['fusion']['b'], params['fusion2']['b']]),
          jnp.stack([params['fusion']['a'], params['fusion2']['a']])[:, None]]
    cp = params['clf']
    clf = [cp['W1'], cp['b1'][None], jnp.reshape(cp['a'], (1, 1)),
           cp['W2'], cp['b2'][None]]

    return _tail(h, fcw_all, aw, fu, clf)


# speedup vs baseline: 3.5762x; 1.0165x over previous
import functools

import jax
import jax.numpy as jnp
from jax import lax
from jax.experimental import pallas as pl
from jax.experimental.pallas import tpu as pltpu
from jax.experimental.pallas import tpu_sc as plsc

D = 128
N_MAIN = 10000
NM_P = 10240
NB = 8192
ROW_B1 = NM_P
ROW_B2 = NM_P + NB
N_H = NM_P + 2 * NB
E_MAIN = 160000
E_MAIN_P = 163840
EB = 65536
E_ALL = E_MAIN_P + 2 * EB
BLOCK = 512
K_SUB = 16

_SC_NC = 2
_SC_NS = 16

_NAGG = 10112

_PHASES = (
    (0, E_MAIN_P, (110, 50), _NAGG, 0, True, 128),
    (E_MAIN_P, EB, (44, 20), NB, ROW_B1, False, 256),
    (E_MAIN_P + EB, EB, (44, 20), NB, ROW_B2, False, 256),
)
_NSP = _NAGG


_CH = 64


def _edge_sc(p_mat, q_mat, sd_pack):
    mesh = plsc.VectorSubcoreMesh(
        core_axis_name="c", subcore_axis_name="s",
        num_cores=_SC_NC, num_subcores=_SC_NS)

    @functools.partial(
        pl.kernel,
        out_type=jax.ShapeDtypeStruct((_SC_NC, N_H, D), jnp.float32),
        mesh=mesh,
        scratch_types=[
            pltpu.VMEM((2, _CH), jnp.int32),
            pltpu.VMEM((2, _CH), jnp.int32),
            pltpu.VMEM((_CH,), jnp.int32),
            pltpu.VMEM((_CH,), jnp.int32),
            pltpu.VMEM((_CH, D), jnp.float32),
            pltpu.VMEM((_CH, D), jnp.float32),
            pltpu.VMEM((_CH, D), jnp.float32),
            pltpu.VMEM((_CH, D), jnp.float32),
            pltpu.VMEM((_CH, D), jnp.float32),
            pltpu.VMEM((_CH, D), jnp.float32),
            pltpu.VMEM_SHARED((_NSP, D), jnp.float32),
            pltpu.SemaphoreType.DMA,
            pltpu.SemaphoreType.DMA,
            pltpu.SemaphoreType.DMA,
            pltpu.SemaphoreType.DMA,
            pltpu.SemaphoreType.DMA,
            pltpu.SemaphoreType.DMA,
            pltpu.SemaphoreType.DMA,
            pltpu.SemaphoreType.DMA,
        ],
    )
    def k(p_hbm, q_hbm, sd_hbm, out_hbm,
          sd0, sd1, dc0, dc1, mr0, mr1, qr0, qr1, sb0, sb1, agg,
          is0, is1, qs0, qs1, gs0, gs1, ss0, ss1):
        c = lax.axis_index("c")
        s = lax.axis_index("s")
        w = s * _SC_NC + c
        sd = (sd0, sd1)
        dc = (dc0, dc1)
        mr = (mr0, mr1)
        qr = (qr0, qr1)
        sb = (sb0, sb1)
        isem = (is0, is1)
        qsem = (qs0, qs1)
        gsem = (gs0, gs1)
        ssem = (ss0, ss1)

        def drain(sem, dst_ref, dummy_src):
            pltpu.make_async_copy(dummy_src, dst_ref, sem).wait()

        def zero_qr0():
            def zb(r, carry):
                for j in range(D // 16):
                    qr0[r, pl.ds(16 * j, 16)] = jnp.zeros((16,), jnp.float32)
                return carry
            lax.fori_loop(0, _CH, zb, 0)

        first = True
        for (e0, cpe, (cpw0, cpw1), n_loc, row_base, has_q, dch) in _PHASES:
            assert 16 * (cpw0 + cpw1) * _CH == cpe
            cpw = jnp.where(c == 0, cpw0, cpw1)
            cbase = jnp.where(c == 0, s * cpw0, 16 * cpw0 + s * cpw1)
            nch = n_loc // _CH
            ch0 = e0 // _CH

            if first or not has_q:
                zero_qr0()
            first = False

            nz = (nch - s + _SC_NS - 1) // _SC_NS

            def zc(i, carry):
                cid = s + i * _SC_NS
                pltpu.sync_copy(qr0, agg.at[pl.ds(cid * _CH, _CH)])
                return carry
            lax.fori_loop(0, nz, zc, 0)
            plsc.subcore_barrier()

            def fire_idx(kk, b):
                cid = ch0 + cbase + kk
                pltpu.async_copy(sd_hbm.at[cid], sd[b], isem[b])

            def fire_q(kk, b):
                cid = ch0 + cbase + kk
                pltpu.async_copy(q_hbm.at[pl.ds(cid * _CH, _CH)],
                                 qr[b], qsem[b])

            def fire_gather(b):
                drain(isem[b], sd[b], sd_hbm.at[0])
                pltpu.async_copy(p_hbm.at[sd[b].at[0]], mr[b], gsem[b])

            def step(kk, b, drain_ssem, prefetch):
                drain(gsem[b], mr[b], p_hbm.at[pl.ds(0, _CH)])
                if drain_ssem:
                    drain(ssem[b], sb[b], p_hbm.at[pl.ds(0, _CH)])
                for j in range(_CH // 16):
                    dc[b][pl.ds(16 * j, 16)] = sd[b][1, pl.ds(16 * j, 16)]
                if prefetch:
                    fire_idx(kk + 2, b)
                if has_q:
                    drain(qsem[b], qr[b], p_hbm.at[pl.ds(0, _CH)])

                def cb(r, carry):
                    for u in range(4):
                        for j in range(D // 16):
                            sl = pl.ds(16 * j, 16)
                            v = mr[b][4 * r + u, sl]
                            if has_q:
                                v = v + qr[b][4 * r + u, sl]
                            sb[b][4 * r + u, sl] = jnp.maximum(v, 0.0)
                    return carry
                lax.fori_loop(0, _CH // 4, cb, 0)

                pltpu.async_copy(sb[b], agg.at[dc[b]], ssem[b], add=True)
                if prefetch:
                    if has_q:
                        fire_q(kk + 2, b)
                    fire_gather(b)

            fire_idx(0, 0)
            fire_idx(1, 1)
            if has_q:
                fire_q(0, 0)
                fire_q(1, 1)
            fire_gather(0)
            fire_gather(1)
            step(0, 0, False, True)
            step(1, 1, False, True)

            def ms(i, carry):
                step(2 + 2 * i, 0, True, True)
                step(3 + 2 * i, 1, True, True)
                return carry
            lax.fori_loop(0, (cpw - 4) // 2, ms, 0)

            step(cpw - 2, 0, True, False)
            step(cpw - 1, 1, True, False)
            drain(ssem[0], sb[0], p_hbm.at[pl.ds(0, _CH)])
            drain(ssem[1], sb[1], p_hbm.at[pl.ds(0, _CH)])
            plsc.subcore_barrier()

            nu = n_loc // dch

            def co(i, carry):
                cid = s + i * _SC_NS
                pltpu.sync_copy(
                    agg.at[pl.ds(cid * dch, dch)],
                    out_hbm.at[c, pl.ds(row_base + cid * dch, dch)])
                return carry
            lax.fori_loop(0, (nu - s + _SC_NS - 1) // _SC_NS, co, 0)
            plsc.subcore_barrier()

    return k(p_mat, q_mat, sd_pack)


def _mm_relu_body(x_ref, w_ref, b_ref, o_ref):
    o_ref[:] = jnp.maximum(
        jnp.dot(x_ref[:], w_ref[:], preferred_element_type=jnp.float32)
        + b_ref[:], 0.0)


def _mm_relu(x, w, b):
    m = x.shape[0]
    return pl.pallas_call(
        _mm_relu_body,
        out_shape=jax.ShapeDtypeStruct((m, D), jnp.float32),
        grid=(m // 512,),
        in_specs=[
            pl.BlockSpec((512, x.shape[1]), lambda i: (i, 0)),
            pl.BlockSpec((x.shape[1], D), lambda i: (0, 0)),
            pl.BlockSpec((1, D), lambda i: (0, 0)),
        ],
        out_specs=pl.BlockSpec((512, D), lambda i: (i, 0)),
    )(x, w, b)


def _mm_bias_body(x_ref, w_ref, b_ref, o_ref):
    o_ref[:] = (jnp.dot(x_ref[:], w_ref[:], preferred_element_type=jnp.float32)
                + b_ref[:])


def _mm_bias_dep_body(x_ref, w_ref, b_ref, d_ref, o_ref):
    del d_ref
    o_ref[:] = (jnp.dot(x_ref[:], w_ref[:], preferred_element_type=jnp.float32)
                + b_ref[:])


def _q_proj(ea, w, b, dep=None):
    m = ea.shape[0]
    ins = [ea, w, b]
    body = _mm_bias_body
    in_specs = [
        pl.BlockSpec((2048, ea.shape[1]), lambda i: (i, 0)),
        pl.BlockSpec((ea.shape[1], D), lambda i: (0, 0)),
        pl.BlockSpec((1, D), lambda i: (0, 0)),
    ]
    if dep is not None:
        ins.append(dep)
        body = _mm_bias_dep_body
        in_specs.append(pl.BlockSpec((1, D), lambda i: (0, 0)))
    return pl.pallas_call(
        body,
        out_shape=jax.ShapeDtypeStruct((m, D), jnp.float32),
        grid=(m // 2048,),
        in_specs=in_specs,
        out_specs=pl.BlockSpec((2048, D), lambda i: (i, 0)),
    )(*ins)


def _p_body(h_ref, w_ref, qt_ref, o_ref):
    o_ref[:] = (jnp.dot(h_ref[:], w_ref[:], preferred_element_type=jnp.float32)
                + qt_ref[0])


def _qtab_sel(i):
    b1 = ROW_B1 // 512
    b2 = ROW_B2 // 512
    return jnp.where(i < b1, 0, jnp.where(i < b2, 1, 2))


def _p_proj(h, w, qtab):
    return pl.pallas_call(
        _p_body,
        out_shape=jax.ShapeDtypeStruct((N_H, D), jnp.float32),
        grid=(N_H // 512,),
        in_specs=[
            pl.BlockSpec((512, D), lambda i: (i, 0)),
            pl.BlockSpec((D, D), lambda i: (0, 0)),
            pl.BlockSpec((1, 512, D), lambda i: (_qtab_sel(i), 0, 0)),
        ],
        out_specs=pl.BlockSpec((512, D), lambda i: (i, 0)),
    )(h, w, qtab)


def _upd_body(h_ref, ap_ref, wh_ref, wa_ref, b_ref, o_ref):
    a = ap_ref[0] + ap_ref[1]
    o_ref[:] = jnp.maximum(
        jnp.dot(h_ref[:], wh_ref[:], preferred_element_type=jnp.float32)
        + jnp.dot(a, wa_ref[:], preferred_element_type=jnp.float32)
        + b_ref[:], 0.0)


def _update(h, aggp, wh, wa, b):
    return pl.pallas_call(
        _upd_body,
        out_shape=jax.ShapeDtypeStruct((N_H, D), jnp.float32),
        grid=(N_H // 512,),
        in_specs=[
            pl.BlockSpec((512, D), lambda i: (i, 0)),
            pl.BlockSpec((2, 512, D), lambda i: (0, i, 0)),
            pl.BlockSpec((D, D), lambda i: (0, 0)),
            pl.BlockSpec((D, D), lambda i: (0, 0)),
            pl.BlockSpec((1, D), lambda i: (0, 0)),
        ],
        out_specs=pl.BlockSpec((512, D), lambda i: (i, 0)),
    )(h, aggp, wh, wa, b)


def _ln(x, s, b):
    m = jnp.mean(x, axis=-1, keepdims=True)
    v = jnp.mean((x - m) ** 2, axis=-1, keepdims=True)
    return (x - m) * lax.rsqrt(v + 1e-5) * s + b


def _dot(a, b):
    return jnp.dot(a, b, preferred_element_type=jnp.float32)


def _tail_body(h_ref, fcw_ref,
               wq_ref, wk_ref, wv_ref, wo_ref,
               bq_ref, bk_ref, bv_ref, bo_ref,
               l1s_ref, l1b_ref, l2s_ref, l2b_ref,
               w1_ref, b1_ref, w2_ref, b2_ref,
               fuw_ref, fub_ref, fua_ref,
               cw1_ref, cb1_ref, ca_ref, cw2_ref, cb2_ref,
               o_ref):
    wgt = h_ref[:] * fcw_ref[:]
    me = jnp.sum(wgt[0:N_MAIN], axis=0, keepdims=True)
    ap1 = jnp.sum(wgt[ROW_B1:ROW_B1 + NB].reshape(K_SUB, BLOCK, D), axis=1)
    ap2 = jnp.sum(wgt[ROW_B2:ROW_B2 + NB].reshape(K_SUB, BLOCK, D), axis=1)

    def prelu(x, a):
        return jnp.where(x >= 0, x, a * x)

    def mha(q, kv, l):
        lq = q.shape[0]
        qp = _dot(q, wq_ref[l]) + bq_ref[l]
        kp = _dot(kv, wk_ref[l]) + bk_ref[l]
        vp = _dot(kv, wv_ref[l]) + bv_ref[l]
        outs = []
        for hh in range(8):
            sl = slice(16 * hh, 16 * hh + 16)
            att = lax.dot_general(qp[:, sl], kp[:, sl],
                                  (((1,), (1,)), ((), ())),
                                  preferred_element_type=jnp.float32) * 0.25
            att = att - jnp.max(att, axis=-1, keepdims=True)
            att = jnp.exp(att)
            att = att / jnp.sum(att, axis=-1, keepdims=True)
            outs.append(_dot(att, vp[:, sl]))
        o = jnp.concatenate(outs, axis=1)
        return _dot(o, wo_ref[l]) + bo_ref[l]

    def enc(x, kv, l):
        x = _ln(x + mha(x, kv, l), l1s_ref[l], l1b_ref[l])
        f = _dot(jnp.maximum(_dot(x, w1_ref[l]) + b1_ref[l], 0.0), w2_ref[l])
        return _ln(x + f + b2_ref[l], l2s_ref[l], l2b_ref[l])

    def branch(ap, g, lbase):
        fw = fuw_ref[g]
        ap = prelu(_dot(ap, fw[:D]) + _dot(me, fw[D:]) + fub_ref[g],
                   fua_ref[g])
        ap = enc(ap, ap, lbase)
        ap = enc(ap, ap, lbase + 1)
        q = enc(me, ap, lbase + 2)
        q = enc(q, ap, lbase + 3)
        return q

    c1 = branch(ap1, 0, 0)
    c2 = branch(ap2, 1, 4)
    ci = jnp.concatenate([me, c1, c2], axis=1)
    z = prelu(_dot(ci, cw1_ref[:]) + cb1_ref[:], ca_ref[0])
    z = _dot(z, cw2_ref[:]) + cb2_ref[:]
    o_ref[:] = 1.0 / (1.0 + jnp.exp(-z))


def _tail(h, fcw, aw, fu, clf):
    ins = [h, fcw] + aw + fu + clf
    return pl.pallas_call(
        _tail_body,
        out_shape=jax.ShapeDtypeStruct((1, 256), jnp.float32),
    )(*ins)


def kernel(params, x_main, ei_main, ea_main, fcw_main, batch_main,
           x1, ei1, ea1, fcw1, batch1, x2, ei2, ea2, fcw2, batch2):
    gnn = params['gnn']
    f32 = jnp.float32

    zpad_m = jnp.zeros((NM_P - N_MAIN, D), f32)
    x_all = jnp.concatenate([x_main, zpad_m, x1, x2], axis=0)
    fcw_all = jnp.concatenate([
        fcw_main, jnp.zeros((NM_P - N_MAIN,), f32), fcw1, fcw2])[:, None]

    ea_all = jnp.concatenate([
        ea_main, ea1[:BLOCK], ea2[:BLOCK],
        jnp.zeros((E_MAIN_P - E_MAIN - 2 * BLOCK, ea_main.shape[1]), f32)],
        axis=0)

    pad_e = E_MAIN_P - E_MAIN
    src_g = jnp.concatenate([
        ei_main[0], jnp.zeros((pad_e,), jnp.int32),
        ei1[0] + ROW_B1, ei2[0] + ROW_B2])
    dst_l = jnp.concatenate([
        ei_main[1], jnp.full((pad_e,), N_MAIN, jnp.int32),
        ei1[1], ei2[1]])
    sd_pack = jnp.stack(
        [src_g.reshape(-1, _CH), dst_l.reshape(-1, _CH)], axis=1)

    h = _mm_relu(x_all, gnn['W0'], gnn['b0'][None])

    prev_p = None
    for lp in gnn['layers']:
        wm, bm = lp['Wm'], lp['bm']
        q_mat = _q_proj(ea_all, wm[D:], bm[None],
                        None if prev_p is None else prev_p[:1])
        qtab = jnp.stack([
            jnp.zeros((BLOCK, D), f32),
            q_mat[E_MAIN:E_MAIN + BLOCK],
            q_mat[E_MAIN + BLOCK:E_MAIN + 2 * BLOCK]])
        p_mat = _p_proj(h, wm[:D], qtab)
        aggp = _edge_sc(p_mat, q_mat, sd_pack)
        h = _update(h, aggp, lp['Wu'][:D], lp['Wu'][D:], lp['bu'][None])
        prev_p = p_mat

    layers = (params['sa1'] + params['cr1'] + params['sa2'] + params['cr2'])

    def stack(name):
        return jnp.stack([l[name] for l in layers])

    aw = [stack('Wq'), stack('Wk'), stack('Wv'), stack('Wo'),
          stack('bq'), stack('bk'), stack('bv'), stack('bo'),
          stack('ln1_s'), stack('ln1_b'), stack('ln2_s'), stack('ln2_b'),
          stack('W1'), stack('b1'), stack('W2'), stack('b2')]
    fu = [jnp.stack([params['fusion']['W'], params['fusion2']['W']]),
          jnp.stack([params['fusion']['b'], params['fusion2']['b']]),
          jnp.stack([params['fusion']['a'], params['fusion2']['a']])[:, None]]
    cp = params['clf']
    clf = [cp['W1'], cp['b1'][None], jnp.reshape(cp['a'], (1, 1)),
           cp['W2'], cp['b2'][None]]

    return _tail(h, fcw_all, aw, fu, clf)
